# Initial kernel scaffold; baseline (speedup 1.0000x reference)
import jax
import jax.numpy as jnp
from jax.experimental import pallas as pl


def kernel(x, edge_weight, params, edge_index):
    raise NotImplementedError("write your pallas kernel here")


# speedup vs baseline: 1.0001x; 1.0001x over previous
import jax
import jax.numpy as jnp
from jax.experimental import pallas as pl
from jax.experimental.pallas import tpu as pltpu


def _bn(h, g, b, eps=1e-5):
    mu = jnp.mean(h, axis=0)
    var = jnp.var(h, axis=0)
    return g * (h - mu) / jnp.sqrt(var + eps) + b


def _gcn(h, row, col, ew, W, b, n):
    deg = jax.ops.segment_sum(ew, col, num_segments=n)
    dis = jnp.where(deg > 0, 1.0 / jnp.sqrt(jnp.maximum(deg, 1e-12)), 0.0)
    norm = dis[row] * ew * dis[col]
    m = (h @ W.T)[row] * norm[:, None]
    return jax.ops.segment_sum(m, col, num_segments=n) + b


def _cls_kernel(h_ref, w1_ref, b1_ref, g_ref, be_ref, w2_ref, b2_ref, o_ref):
    h = h_ref[...]
    y = jnp.dot(h, w1_ref[...].T, preferred_element_type=jnp.float32) + b1_ref[...]
    mu = jnp.mean(y, axis=0)
    var = jnp.mean((y - mu) ** 2, axis=0)
    y = g_ref[...] * (y - mu) / jnp.sqrt(var + 1e-5) + be_ref[...]
    y = jnp.maximum(y, 0.0)
    o_ref[...] = jnp.dot(y, w2_ref[...].T, preferred_element_type=jnp.float32) + b2_ref[...]


def _classifier(hstate, p):
    N = hstate.shape[0]
    C = p['cls_b2'].shape[0]
    out = pl.pallas_call(
        _cls_kernel,
        out_shape=jax.ShapeDtypeStruct((N, 128), jnp.float32),
    )(hstate, p['cls_w1'], p['cls_b1'], p['cls_g'], p['cls_be'],
      jnp.pad(p['cls_w2'], ((0, 128 - C), (0, 0))),
      jnp.pad(p['cls_b2'], (0, 128 - C)))
    return out[:, :C]


def kernel(x, edge_weight, params, edge_index):
    p = params
    T, N, _ = x.shape
    loop = jnp.arange(N)
    row = jnp.concatenate([edge_index[0], loop])
    col = jnp.concatenate([edge_index[1], loop])
    ew = jnp.concatenate([edge_weight, jnp.ones((N,), x.dtype)])
    steps = []
    for t in range(T):
        h = jax.nn.relu(_bn(x[t] @ p['pre_w1'].T + p['pre_b1'], p['pre_g1'], p['pre_be1']))
        h = jax.nn.relu(_bn(h @ p['pre_w2'].T + p['pre_b2'], p['pre_g2'], p['pre_be2']))
        for i in range(3):
            hn = _gcn(h, row, col, ew, p['gcn_w'][i], p['gcn_b'][i], N)
            hn = jax.nn.relu(_bn(hn, p['gbn_g'][i], p['gbn_b'][i]))
            h = h + hn
        h2 = jax.nn.relu(_bn(h @ p['post_w'].T + p['post_b'], p['post_g'], p['post_be']))
        steps.append(h2)
    H = jnp.stack(steps, axis=1)
    hidden = p['gru_whh'].shape[1]
    hstate = jnp.zeros((N, hidden), x.dtype)
    for t in range(T):
        gi = H[:, t, :] @ p['gru_wih'].T + p['gru_bih']
        gh = hstate @ p['gru_whh'].T + p['gru_bhh']
        ir, iz, i_n = jnp.split(gi, 3, axis=1)
        hr, hz, h_n = jnp.split(gh, 3, axis=1)
        r = jax.nn.sigmoid(ir + hr)
        z = jax.nn.sigmoid(iz + hz)
        n_ = jnp.tanh(i_n + r * h_n)
        hstate = (1.0 - z) * n_ + z * hstate
    return _classifier(hstate, p)


# speedup vs baseline: 4.0814x; 4.0808x over previous
import functools

import jax
import jax.numpy as jnp
from jax import lax
from jax.experimental import pallas as pl
from jax.experimental.pallas import tpu as pltpu
from jax.experimental.pallas import tpu_sc as plsc

N = 10000
T = 8
E = 320000
E_SL = E + N
NTILES = 32
SUB = 16
CHUNK = 128
EPT = 10368
E_PAD = EPT * NTILES
NP = 10240
ROWS_PER_SUB = NP // SUB


def _make_spmm(nt):
    mesh = plsc.VectorSubcoreMesh(core_axis_name="c", subcore_axis_name="s")

    @functools.partial(
        pl.kernel,
        mesh=mesh,
        out_type=jax.ShapeDtypeStruct((2, nt, NP, 128), jnp.float32),
        scratch_types=[
            pltpu.VMEM((CHUNK,), jnp.int32),
            pltpu.VMEM((CHUNK,), jnp.int32),
            pltpu.VMEM((CHUNK,), jnp.int32),
            pltpu.VMEM((CHUNK, 16), jnp.float32),
            pltpu.VMEM((CHUNK, 128), jnp.float32),
            pltpu.VMEM_SHARED((NP, 128), jnp.float32),
            pltpu.SemaphoreType.DMA,
        ],
    )
    def spmm(table_hbm, row_hbm, col_hbm, ew_hbm, zeros_hbm, out_hbm,
             ridx_v, radj_v, cidx_v, ew_v, rows_v, acc, sem):
        c = lax.axis_index("c")
        s = lax.axis_index("s")
        wid = s * 2 + c
        ebase = wid * EPT

        for t in range(nt):
            pltpu.sync_copy(zeros_hbm, acc.at[pl.ds(s * ROWS_PER_SUB, ROWS_PER_SUB)])
            plsc.subcore_barrier()

            def chunk_body(jc, carry):
                base = ebase + jc * CHUNK
                pltpu.sync_copy(row_hbm.at[pl.ds(base, CHUNK)], ridx_v)
                pltpu.sync_copy(col_hbm.at[pl.ds(base, CHUNK)], cidx_v)
                pltpu.sync_copy(ew_hbm.at[pl.ds(base, CHUNK)], ew_v)
                off = jnp.full((16,), t * N, jnp.int32)
                for g in range(CHUNK // 16):
                    radj_v[pl.ds(g * 16, 16)] = ridx_v[pl.ds(g * 16, 16)] + off
                pltpu.async_copy(table_hbm.at[radj_v], rows_v, sem).wait()

                def edge_body(e, carry2):
                    bc = ew_v[e, :]
                    for j in range(8):
                        rows_v[e, pl.ds(j * 16, 16)] = (
                            rows_v[e, pl.ds(j * 16, 16)] * bc)
                    return carry2
                lax.fori_loop(0, CHUNK, edge_body, 0, unroll=2)

                pltpu.sync_copy(rows_v, acc.at[cidx_v], add=True)
                return carry
            lax.fori_loop(0, EPT // CHUNK, chunk_body, 0)

            plsc.subcore_barrier()
            pltpu.sync_copy(
                acc.at[pl.ds(s * ROWS_PER_SUB, ROWS_PER_SUB)],
                out_hbm.at[c, t, pl.ds(s * ROWS_PER_SUB, ROWS_PER_SUB)])
            plsc.subcore_barrier()

    return spmm


_spmm_T = _make_spmm(T)
_spmm_1 = _make_spmm(1)


def _bn_relu(y, g, b, n_rows):
    mu = jnp.sum(y, axis=0, keepdims=True) / n_rows
    var = jnp.sum((y - mu) ** 2, axis=0, keepdims=True) / n_rows
    return jnp.maximum(g * (y - mu) / jnp.sqrt(var + 1e-5) + b, 0.0)


def _dis_from_parts(p0, p1):
    return lax.rsqrt(p0 + p1)


def _pre_body(x_ref, p0_ref, p1_ref, w1_ref, b1_ref, g1_ref, be1_ref,
              w2_ref, b2_ref, g2_ref, be2_ref, gw_ref,
              h_ref, a_ref):
    x = x_ref[0]
    h = jnp.dot(x, w1_ref[...].T, preferred_element_type=jnp.float32) + b1_ref[...]
    h = _bn_relu(h, g1_ref[...], be1_ref[...], N)
    h = jnp.dot(h, w2_ref[...].T, preferred_element_type=jnp.float32) + b2_ref[...]
    h = _bn_relu(h, g2_ref[...], be2_ref[...], N)
    h_ref[0] = h
    dis = _dis_from_parts(p0_ref[...], p1_ref[...])
    a_ref[0] = dis * jnp.dot(h, gw_ref[...].T, preferred_element_type=jnp.float32)


def _resid_body(sp_ref, h_ref, p0_ref, p1_ref, gb_ref, g_ref, be_ref,
                h_out_ref):
    dis = _dis_from_parts(p0_ref[...], p1_ref[...])
    srow = sp_ref[0, 0, :N] + sp_ref[1, 0, :N]
    hn = dis * srow + gb_ref[...]
    hn = _bn_relu(hn, g_ref[...], be_ref[...], N)
    h_out_ref[0] = h_ref[0] + hn


def _table_body(h_ref, p0_ref, p1_ref, w_ref, a_ref):
    dis = _dis_from_parts(p0_ref[...], p1_ref[...])
    a_ref[0] = dis * jnp.dot(h_ref[0], w_ref[...].T,
                             preferred_element_type=jnp.float32)


def _postmlp_body(h_ref, w_ref, b_ref, g_ref, be_ref, out_ref):
    y = jnp.dot(h_ref[0], w_ref[...].T, preferred_element_type=jnp.float32)
    out_ref[0] = _bn_relu(y + b_ref[...], g_ref[...], be_ref[...], N)


def _gru_body(H_ref, wih_ref, whh_ref, bih_ref, bhh_ref,
              w1_ref, b1_ref, g_ref, be_ref, w2_ref, b2_ref,
              out_ref, hstate):
    t = pl.program_id(0)

    @pl.when(t == 0)
    def _():
        hstate[...] = jnp.zeros((N, 128), jnp.float32)

    h = H_ref[0]
    gi = jnp.dot(h, wih_ref[...].T, preferred_element_type=jnp.float32) + bih_ref[...]
    gh = jnp.dot(hstate[...], whh_ref[...].T, preferred_element_type=jnp.float32) + bhh_ref[...]
    r = jax.nn.sigmoid(gi[:, :128] + gh[:, :128])
    z = jax.nn.sigmoid(gi[:, 128:256] + gh[:, 128:256])
    n_ = jnp.tanh(gi[:, 256:] + r * gh[:, 256:])
    hs = (1.0 - z) * n_ + z * hstate[...]
    hstate[...] = hs

    @pl.when(t == T - 1)
    def _():
        y = jnp.dot(hs, w1_ref[...].T, preferred_element_type=jnp.float32) + b1_ref[...]
        y = _bn_relu(y, g_ref[...], be_ref[...], N)
        out_ref[...] = jnp.dot(y, w2_ref[...].T, preferred_element_type=jnp.float32) + b2_ref[...]


def _full(shape):
    return pl.BlockSpec(shape, lambda t: tuple(0 for _ in shape))


def _per_t(shape):
    return pl.BlockSpec(shape, lambda t: (t,) + tuple(0 for _ in shape[1:]))


def kernel(x, edge_weight, params, edge_index):
    p = params
    f32 = jnp.float32

    npad = E_PAD - E_SL
    sl = jnp.arange(N, dtype=jnp.int32)
    row = jnp.concatenate(
        [edge_index[0], sl, (jnp.arange(npad, dtype=jnp.int32) * 37) % N])
    col = jnp.concatenate([edge_index[1], sl, jnp.zeros((npad,), jnp.int32)])
    ew = jnp.concatenate(
        [edge_weight.astype(f32), jnp.ones((N,), f32), jnp.zeros((npad,), f32)])
    ew_wide = jnp.broadcast_to(ew[:, None], (E_PAD, 16))
    zeros_sub = jnp.zeros((ROWS_PER_SUB, 128), f32)

    degp = _spmm_1(jnp.ones((N, 128), f32), row, col, ew_wide, zeros_sub)
    dp0 = degp[0, 0, :N, 0:1]
    dp1 = degp[1, 0, :N, 0:1]

    r2 = lambda v: v.reshape(1, -1)

    h0, a0 = pl.pallas_call(
        _pre_body,
        grid=(T,),
        in_specs=[
            _per_t((1, N, 128)),
            _full((N, 1)), _full((N, 1)),
            _full((256, 128)), _full((1, 256)), _full((1, 256)), _full((1, 256)),
            _full((128, 256)), _full((1, 128)), _full((1, 128)), _full((1, 128)),
            _full((128, 128)),
        ],
        out_specs=[_per_t((1, N, 128)), _per_t((1, N, 128))],
        out_shape=[jax.ShapeDtypeStruct((T, N, 128), f32),
                   jax.ShapeDtypeStruct((T, N, 128), f32)],
    )(x, dp0, dp1,
      p['pre_w1'], r2(p['pre_b1']), r2(p['pre_g1']), r2(p['pre_be1']),
      p['pre_w2'], r2(p['pre_b2']), r2(p['pre_g2']), r2(p['pre_be2']),
      p['gcn_w'][0])

    h, a = h0, a0
    for i in range(3):
        sp = _spmm_T(a.reshape(T * N, 128), row, col, ew_wide, zeros_sub)
        h = pl.pallas_call(
            _resid_body,
            grid=(T,),
            in_specs=[
                pl.BlockSpec((2, 1, NP, 128), lambda t: (0, t, 0, 0)),
                _per_t((1, N, 128)),
                _full((N, 1)), _full((N, 1)),
                _full((1, 128)), _full((1, 128)), _full((1, 128)),
            ],
            out_specs=_per_t((1, N, 128)),
            out_shape=jax.ShapeDtypeStruct((T, N, 128), f32),
        )(sp, h, dp0, dp1,
          r2(p['gcn_b'][i]), r2(p['gbn_g'][i]), r2(p['gbn_b'][i]))
        if i < 2:
            a = pl.pallas_call(
                _table_body,
                grid=(T,),
                in_specs=[_per_t((1, N, 128)), _full((N, 1)), _full((N, 1)),
                          _full((128, 128))],
                out_specs=_per_t((1, N, 128)),
                out_shape=jax.ShapeDtypeStruct((T, N, 128), f32),
            )(h, dp0, dp1, p['gcn_w'][i + 1])

    H = pl.pallas_call(
        _postmlp_body,
        grid=(T,),
        in_specs=[_per_t((1, N, 128)), _full((128, 128)),
                  _full((1, 128)), _full((1, 128)), _full((1, 128))],
        out_specs=_per_t((1, N, 128)),
        out_shape=jax.ShapeDtypeStruct((T, N, 128), f32),
    )(h, p['post_w'], r2(p['post_b']), r2(p['post_g']), r2(p['post_be']))

    out = pl.pallas_call(
        _gru_body,
        grid=(T,),
        in_specs=[
            _per_t((1, N, 128)),
            _full((384, 128)), _full((384, 128)), _full((1, 384)), _full((1, 384)),
            _full((256, 128)), _full((1, 256)), _full((1, 256)), _full((1, 256)),
            _full((128, 256)), _full((1, 128)),
        ],
        out_specs=_full((N, 128)),
        out_shape=jax.ShapeDtypeStruct((N, 128), f32),
        scratch_shapes=[pltpu.VMEM((N, 128), f32)],
    )(H, p['gru_wih'], p['gru_whh'], r2(p['gru_bih']), r2(p['gru_bhh']),
      p['cls_w1'], r2(p['cls_b1']), r2(p['cls_g']), r2(p['cls_be']),
      jnp.pad(p['cls_w2'], ((0, 128 - p['cls_w2'].shape[0]), (0, 0))),
      r2(jnp.pad(p['cls_b2'], (0, 128 - p['cls_b2'].shape[0]))))

    return out[:, :p['cls_b2'].shape[0]]


# speedup vs baseline: 5.4598x; 1.3377x over previous
import functools

import jax
import jax.numpy as jnp
from jax import lax
from jax.experimental import pallas as pl
from jax.experimental.pallas import tpu as pltpu
from jax.experimental.pallas import tpu_sc as plsc

N = 10000
T = 8
E = 320000
E_SL = E + N
NTILES = 32
SUB = 16
CHUNK = 64
NCH = 164
EPT = NCH * CHUNK
E_PAD = EPT * NTILES
E_ALL = E_PAD + CHUNK
NP = 10240
ROWS_PER_SUB = NP // SUB


def _make_spmm(nt):
    mesh = plsc.VectorSubcoreMesh(core_axis_name="c", subcore_axis_name="s")

    @functools.partial(
        pl.kernel,
        mesh=mesh,
        out_type=jax.ShapeDtypeStruct((2, nt, NP, 128), jnp.float32),
        scratch_types=[
            pltpu.VMEM((CHUNK,), jnp.int32),
            pltpu.VMEM((CHUNK,), jnp.int32),
            pltpu.VMEM((CHUNK,), jnp.int32),
            pltpu.VMEM((CHUNK,), jnp.int32),
            pltpu.VMEM((CHUNK, 16), jnp.float32),
            pltpu.VMEM((CHUNK, 16), jnp.float32),
            pltpu.VMEM((CHUNK, 128), jnp.float32),
            pltpu.VMEM((CHUNK, 128), jnp.float32),
            pltpu.VMEM_SHARED((NP, 128), jnp.float32),
            pltpu.SemaphoreType.DMA,
            pltpu.SemaphoreType.DMA,
        ],
    )
    def spmm(table_hbm, radj_hbm, col_hbm, ew_hbm, zeros_hbm, out_hbm,
             riA, riB, ciA, ciB, ewA, ewB, rowsA, rowsB, acc,
             sem_g, sem_i):
        c = lax.axis_index("c")
        s = lax.axis_index("s")
        wid = s * 2 + c
        ebase = wid * EPT

        def load_idx_sync(t, j, ri, ci, ew):
            base = ebase + j * CHUNK
            pltpu.sync_copy(radj_hbm.at[t, pl.ds(base, CHUNK)], ri)
            pltpu.sync_copy(col_hbm.at[pl.ds(base, CHUNK)], ci)
            pltpu.sync_copy(ew_hbm.at[pl.ds(base, CHUNK)], ew)

        def mul(rows_v, ew_v):
            def edge_body(e, carry2):
                bc = ew_v[e, :]
                for g in range(8):
                    rows_v[e, pl.ds(g * 16, 16)] = (
                        rows_v[e, pl.ds(g * 16, 16)] * bc)
                return carry2
            lax.fori_loop(0, CHUNK, edge_body, 0, unroll=2)

        def step(t, j, cur, nxt, prefetch):
            (ri0, ci0, ew0, rows0) = cur
            (ri1, ci1, ew1, rows1) = nxt
            if prefetch:
                hg = pltpu.async_copy(table_hbm.at[ri1], rows1, sem_g)
            mul(rows0, ew0)
            pltpu.sync_copy(rows0, acc.at[ci0], add=True)
            if prefetch:
                base2 = ebase + (j + 2) * CHUNK
                h1 = pltpu.async_copy(radj_hbm.at[t, pl.ds(base2, CHUNK)], ri0, sem_i)
                h2 = pltpu.async_copy(col_hbm.at[pl.ds(base2, CHUNK)], ci0, sem_i)
                h3 = pltpu.async_copy(ew_hbm.at[pl.ds(base2, CHUNK)], ew0, sem_i)
                hg.wait()
                h1.wait()
                h2.wait()
                h3.wait()

        bufA = (riA, ciA, ewA, rowsA)
        bufB = (riB, ciB, ewB, rowsB)

        def t_body(t, carry):
            pltpu.sync_copy(zeros_hbm, acc.at[pl.ds(s * ROWS_PER_SUB, ROWS_PER_SUB)])
            plsc.subcore_barrier()

            load_idx_sync(t, 0, riA, ciA, ewA)
            pltpu.async_copy(table_hbm.at[riA], rowsA, sem_g).wait()
            load_idx_sync(t, 1, riB, ciB, ewB)

            def pair_body(k, carry2):
                step(t, 2 * k, bufA, bufB, True)
                step(t, 2 * k + 1, bufB, bufA, True)
                return carry2
            lax.fori_loop(0, NCH // 2 - 1, pair_body, 0)
            step(t, NCH - 2, bufA, bufB, True)
            step(t, NCH - 1, bufB, bufA, False)

            plsc.subcore_barrier()
            pltpu.sync_copy(
                acc.at[pl.ds(s * ROWS_PER_SUB, ROWS_PER_SUB)],
                out_hbm.at[c, t, pl.ds(s * ROWS_PER_SUB, ROWS_PER_SUB)])
            plsc.subcore_barrier()
            return carry

        lax.fori_loop(0, nt, t_body, 0)

    return spmm


_spmm_T = _make_spmm(T)
_spmm_1 = _make_spmm(1)


def _bn_relu(y, g, b, n_rows):
    mu = jnp.sum(y, axis=0, keepdims=True) / n_rows
    var = jnp.sum((y - mu) ** 2, axis=0, keepdims=True) / n_rows
    return jnp.maximum(g * (y - mu) / jnp.sqrt(var + 1e-5) + b, 0.0)


def _dis_from_parts(p0, p1):
    return lax.rsqrt(p0 + p1)


def _pre_body(x_ref, p0_ref, p1_ref, w1_ref, b1_ref, g1_ref, be1_ref,
              w2_ref, b2_ref, g2_ref, be2_ref, gw_ref,
              h_ref, a_ref):
    x = x_ref[0]
    h = jnp.dot(x, w1_ref[...].T, preferred_element_type=jnp.float32) + b1_ref[...]
    h = _bn_relu(h, g1_ref[...], be1_ref[...], N)
    h = jnp.dot(h, w2_ref[...].T, preferred_element_type=jnp.float32) + b2_ref[...]
    h = _bn_relu(h, g2_ref[...], be2_ref[...], N)
    h_ref[0] = h
    dis = _dis_from_parts(p0_ref[...], p1_ref[...])
    a_ref[0] = dis * jnp.dot(h, gw_ref[...].T, preferred_element_type=jnp.float32)


def _resid_body(sp_ref, h_ref, p0_ref, p1_ref, gb_ref, g_ref, be_ref,
                h_out_ref):
    dis = _dis_from_parts(p0_ref[...], p1_ref[...])
    srow = sp_ref[0, 0, :N] + sp_ref[1, 0, :N]
    hn = dis * srow + gb_ref[...]
    hn = _bn_relu(hn, g_ref[...], be_ref[...], N)
    h_out_ref[0] = h_ref[0] + hn


def _table_body(h_ref, p0_ref, p1_ref, w_ref, a_ref):
    dis = _dis_from_parts(p0_ref[...], p1_ref[...])
    a_ref[0] = dis * jnp.dot(h_ref[0], w_ref[...].T,
                             preferred_element_type=jnp.float32)


def _postmlp_body(h_ref, w_ref, b_ref, g_ref, be_ref, out_ref):
    y = jnp.dot(h_ref[0], w_ref[...].T, preferred_element_type=jnp.float32)
    out_ref[0] = _bn_relu(y + b_ref[...], g_ref[...], be_ref[...], N)


def _gru_body(H_ref, wih_ref, whh_ref, bih_ref, bhh_ref,
              w1_ref, b1_ref, g_ref, be_ref, w2_ref, b2_ref,
              out_ref, hstate):
    t = pl.program_id(0)

    @pl.when(t == 0)
    def _():
        hstate[...] = jnp.zeros((N, 128), jnp.float32)

    h = H_ref[0]
    gi = jnp.dot(h, wih_ref[...].T, preferred_element_type=jnp.float32) + bih_ref[...]
    gh = jnp.dot(hstate[...], whh_ref[...].T, preferred_element_type=jnp.float32) + bhh_ref[...]
    r = jax.nn.sigmoid(gi[:, :128] + gh[:, :128])
    z = jax.nn.sigmoid(gi[:, 128:256] + gh[:, 128:256])
    n_ = jnp.tanh(gi[:, 256:] + r * gh[:, 256:])
    hs = (1.0 - z) * n_ + z * hstate[...]
    hstate[...] = hs

    @pl.when(t == T - 1)
    def _():
        y = jnp.dot(hs, w1_ref[...].T, preferred_element_type=jnp.float32) + b1_ref[...]
        y = _bn_relu(y, g_ref[...], be_ref[...], N)
        out_ref[...] = jnp.dot(y, w2_ref[...].T, preferred_element_type=jnp.float32) + b2_ref[...]


def _full(shape):
    return pl.BlockSpec(shape, lambda t: tuple(0 for _ in shape))


def _per_t(shape):
    return pl.BlockSpec(shape, lambda t: (t,) + tuple(0 for _ in shape[1:]))


def kernel(x, edge_weight, params, edge_index):
    p = params
    f32 = jnp.float32

    npad = E_ALL - E_SL
    sl = jnp.arange(N, dtype=jnp.int32)
    row = jnp.concatenate(
        [edge_index[0], sl, (jnp.arange(npad, dtype=jnp.int32) * 37) % N])
    col = jnp.concatenate([edge_index[1], sl, jnp.zeros((npad,), jnp.int32)])
    ew = jnp.concatenate(
        [edge_weight.astype(f32), jnp.ones((N,), f32), jnp.zeros((npad,), f32)])
    ew_wide = jnp.broadcast_to(ew[:, None], (E_ALL, 16))
    zeros_sub = jnp.zeros((ROWS_PER_SUB, 128), f32)
    radj1 = row[None, :]
    radj8 = row[None, :] + (jnp.arange(T, dtype=jnp.int32) * N)[:, None]

    degp = _spmm_1(jnp.ones((N, 128), f32), radj1, col, ew_wide, zeros_sub)
    dp0 = degp[0, 0, :N, 0:1]
    dp1 = degp[1, 0, :N, 0:1]

    r2 = lambda v: v.reshape(1, -1)

    h0, a0 = pl.pallas_call(
        _pre_body,
        grid=(T,),
        in_specs=[
            _per_t((1, N, 128)),
            _full((N, 1)), _full((N, 1)),
            _full((256, 128)), _full((1, 256)), _full((1, 256)), _full((1, 256)),
            _full((128, 256)), _full((1, 128)), _full((1, 128)), _full((1, 128)),
            _full((128, 128)),
        ],
        out_specs=[_per_t((1, N, 128)), _per_t((1, N, 128))],
        out_shape=[jax.ShapeDtypeStruct((T, N, 128), f32),
                   jax.ShapeDtypeStruct((T, N, 128), f32)],
    )(x, dp0, dp1,
      p['pre_w1'], r2(p['pre_b1']), r2(p['pre_g1']), r2(p['pre_be1']),
      p['pre_w2'], r2(p['pre_b2']), r2(p['pre_g2']), r2(p['pre_be2']),
      p['gcn_w'][0])

    h, a = h0, a0
    for i in range(3):
        sp = _spmm_T(a.reshape(T * N, 128), radj8, col, ew_wide, zeros_sub)
        h = pl.pallas_call(
            _resid_body,
            grid=(T,),
            in_specs=[
                pl.BlockSpec((2, 1, NP, 128), lambda t: (0, t, 0, 0)),
                _per_t((1, N, 128)),
                _full((N, 1)), _full((N, 1)),
                _full((1, 128)), _full((1, 128)), _full((1, 128)),
            ],
            out_specs=_per_t((1, N, 128)),
            out_shape=jax.ShapeDtypeStruct((T, N, 128), f32),
        )(sp, h, dp0, dp1,
          r2(p['gcn_b'][i]), r2(p['gbn_g'][i]), r2(p['gbn_b'][i]))
        if i < 2:
            a = pl.pallas_call(
                _table_body,
                grid=(T,),
                in_specs=[_per_t((1, N, 128)), _full((N, 1)), _full((N, 1)),
                          _full((128, 128))],
                out_specs=_per_t((1, N, 128)),
                out_shape=jax.ShapeDtypeStruct((T, N, 128), f32),
            )(h, dp0, dp1, p['gcn_w'][i + 1])

    H = pl.pallas_call(
        _postmlp_body,
        grid=(T,),
        in_specs=[_per_t((1, N, 128)), _full((128, 128)),
                  _full((1, 128)), _full((1, 128)), _full((1, 128))],
        out_specs=_per_t((1, N, 128)),
        out_shape=jax.ShapeDtypeStruct((T, N, 128), f32),
    )(h, p['post_w'], r2(p['post_b']), r2(p['post_g']), r2(p['post_be']))

    out = pl.pallas_call(
        _gru_body,
        grid=(T,),
        in_specs=[
            _per_t((1, N, 128)),
            _full((384, 128)), _full((384, 128)), _full((1, 384)), _full((1, 384)),
            _full((256, 128)), _full((1, 256)), _full((1, 256)), _full((1, 256)),
            _full((128, 256)), _full((1, 128)),
        ],
        out_specs=_full((N, 128)),
        out_shape=jax.ShapeDtypeStruct((N, 128), f32),
        scratch_shapes=[pltpu.VMEM((N, 128), f32)],
    )(H, p['gru_wih'], p['gru_whh'], r2(p['gru_bih']), r2(p['gru_bhh']),
      p['cls_w1'], r2(p['cls_b1']), r2(p['cls_g']), r2(p['cls_be']),
      jnp.pad(p['cls_w2'], ((0, 128 - p['cls_w2'].shape[0]), (0, 0))),
      r2(jnp.pad(p['cls_b2'], (0, 128 - p['cls_b2'].shape[0]))))

    return out[:, :p['cls_b2'].shape[0]]


# speedup vs baseline: 6.4937x; 1.1894x over previous
import functools

import jax
import jax.numpy as jnp
from jax import lax
from jax.experimental import pallas as pl
from jax.experimental.pallas import tpu as pltpu
from jax.experimental.pallas import tpu_sc as plsc

N = 10000
T = 8
E = 320000
E_SL = E + N
NTILES = 32
SUB = 16
CHUNK = 64
NCH = 164
EPT = NCH * CHUNK
E_PAD = EPT * NTILES
E_ALL = E_PAD + CHUNK
NP = 10240
ROWS_PER_SUB = NP // SUB


def _make_spmm(nt):
    mesh = plsc.VectorSubcoreMesh(core_axis_name="c", subcore_axis_name="s")

    @functools.partial(
        pl.kernel,
        mesh=mesh,
        out_type=jax.ShapeDtypeStruct((2, nt, NP, 128), jnp.float32),
        scratch_types=[
            pltpu.VMEM((CHUNK,), jnp.int32),
            pltpu.VMEM((CHUNK,), jnp.int32),
            pltpu.VMEM((CHUNK,), jnp.int32),
            pltpu.VMEM((CHUNK,), jnp.int32),
            pltpu.VMEM((CHUNK,), jnp.int32),
            pltpu.VMEM((CHUNK,), jnp.int32),
            pltpu.VMEM((CHUNK, 16), jnp.float32),
            pltpu.VMEM((CHUNK, 16), jnp.float32),
            pltpu.VMEM((CHUNK, 128), jnp.float32),
            pltpu.VMEM((CHUNK, 128), jnp.float32),
            pltpu.VMEM_SHARED((NP, 128), jnp.float32),
            pltpu.SemaphoreType.DMA,
            pltpu.SemaphoreType.DMA,
            pltpu.SemaphoreType.DMA,
            pltpu.SemaphoreType.DMA,
        ],
    )
    def spmm(table_hbm, radj_hbm, col_hbm, ew_hbm, zeros_hbm, out_hbm,
             riA, riB, ciA, ciB, sciA, sciB, ewA, ewB, rowsA, rowsB, acc,
             sem_g, sem_i, sem_sA, sem_sB):
        c = lax.axis_index("c")
        s = lax.axis_index("s")
        wid = s * 2 + c
        ebase = wid * EPT

        def load_idx_sync(t, j, ri, ci, ew):
            base = ebase + j * CHUNK
            pltpu.sync_copy(radj_hbm.at[t, pl.ds(base, CHUNK)], ri)
            pltpu.sync_copy(col_hbm.at[pl.ds(base, CHUNK)], ci)
            pltpu.sync_copy(ew_hbm.at[pl.ds(base, CHUNK)], ew)

        def mul(rows_v, ew_v):
            def edge_body(e, carry2):
                bc = ew_v[e, :]
                for g in range(8):
                    rows_v[e, pl.ds(g * 16, 16)] = (
                        rows_v[e, pl.ds(g * 16, 16)] * bc)
                return carry2
            lax.fori_loop(0, CHUNK, edge_body, 0, unroll=2)

        def step(t, j, cur, nxt, first, last):
            (ri0, ci0, sci0, ew0, rows0, sem_s0) = cur
            (ri1, ci1, sci1, ew1, rows1, sem_s1) = nxt
            if not first:
                pltpu.make_async_copy(rows1, acc.at[sci1], sem_s1).wait()
            if not last:
                hg = pltpu.async_copy(table_hbm.at[ri1], rows1, sem_g)
            mul(rows0, ew0)
            for g in range(CHUNK // 16):
                sci0[pl.ds(g * 16, 16)] = ci0[pl.ds(g * 16, 16)]
            pltpu.async_copy(rows0, acc.at[sci0], sem_s0, add=True)
            if not last:
                base2 = ebase + (j + 2) * CHUNK
                h1 = pltpu.async_copy(radj_hbm.at[t, pl.ds(base2, CHUNK)], ri0, sem_i)
                h2 = pltpu.async_copy(col_hbm.at[pl.ds(base2, CHUNK)], ci0, sem_i)
                h3 = pltpu.async_copy(ew_hbm.at[pl.ds(base2, CHUNK)], ew0, sem_i)
                hg.wait()
                h1.wait()
                h2.wait()
                h3.wait()

        bufA = (riA, ciA, sciA, ewA, rowsA, sem_sA)
        bufB = (riB, ciB, sciB, ewB, rowsB, sem_sB)

        def t_body(t, carry):
            pltpu.sync_copy(zeros_hbm, acc.at[pl.ds(s * ROWS_PER_SUB, ROWS_PER_SUB)])
            plsc.subcore_barrier()

            load_idx_sync(t, 0, riA, ciA, ewA)
            pltpu.async_copy(table_hbm.at[riA], rowsA, sem_g).wait()
            load_idx_sync(t, 1, riB, ciB, ewB)

            step(t, 0, bufA, bufB, True, False)
            step(t, 1, bufB, bufA, False, False)

            def pair_body(k, carry2):
                step(t, 2 * k, bufA, bufB, False, False)
                step(t, 2 * k + 1, bufB, bufA, False, False)
                return carry2
            lax.fori_loop(1, NCH // 2 - 1, pair_body, 0)
            step(t, NCH - 2, bufA, bufB, False, False)
            step(t, NCH - 1, bufB, bufA, False, True)
            pltpu.make_async_copy(rowsB, acc.at[sciB], sem_sB).wait()

            plsc.subcore_barrier()
            pltpu.sync_copy(
                acc.at[pl.ds(s * ROWS_PER_SUB, ROWS_PER_SUB)],
                out_hbm.at[c, t, pl.ds(s * ROWS_PER_SUB, ROWS_PER_SUB)])
            plsc.subcore_barrier()
            return carry

        lax.fori_loop(0, nt, t_body, 0)

    return spmm


_spmm_T = _make_spmm(T)
_spmm_1 = _make_spmm(1)


def _bn_relu(y, g, b, n_rows):
    mu = jnp.sum(y, axis=0, keepdims=True) / n_rows
    var = jnp.sum((y - mu) ** 2, axis=0, keepdims=True) / n_rows
    return jnp.maximum(g * (y - mu) / jnp.sqrt(var + 1e-5) + b, 0.0)


def _dis_from_parts(p0, p1):
    return lax.rsqrt(p0 + p1)


def _pre_body(x_ref, p0_ref, p1_ref, w1_ref, b1_ref, g1_ref, be1_ref,
              w2_ref, b2_ref, g2_ref, be2_ref, gw_ref,
              h_ref, a_ref):
    x = x_ref[0]
    h = jnp.dot(x, w1_ref[...].T, preferred_element_type=jnp.float32) + b1_ref[...]
    h = _bn_relu(h, g1_ref[...], be1_ref[...], N)
    h = jnp.dot(h, w2_ref[...].T, preferred_element_type=jnp.float32) + b2_ref[...]
    h = _bn_relu(h, g2_ref[...], be2_ref[...], N)
    h_ref[0] = h
    dis = _dis_from_parts(p0_ref[...], p1_ref[...])
    a_ref[0] = dis * jnp.dot(h, gw_ref[...].T, preferred_element_type=jnp.float32)


def _resid_body(sp_ref, h_ref, p0_ref, p1_ref, gb_ref, g_ref, be_ref,
                h_out_ref):
    dis = _dis_from_parts(p0_ref[...], p1_ref[...])
    srow = sp_ref[0, 0, :N] + sp_ref[1, 0, :N]
    hn = dis * srow + gb_ref[...]
    hn = _bn_relu(hn, g_ref[...], be_ref[...], N)
    h_out_ref[0] = h_ref[0] + hn


def _table_body(h_ref, p0_ref, p1_ref, w_ref, a_ref):
    dis = _dis_from_parts(p0_ref[...], p1_ref[...])
    a_ref[0] = dis * jnp.dot(h_ref[0], w_ref[...].T,
                             preferred_element_type=jnp.float32)


def _postmlp_body(h_ref, w_ref, b_ref, g_ref, be_ref, out_ref):
    y = jnp.dot(h_ref[0], w_ref[...].T, preferred_element_type=jnp.float32)
    out_ref[0] = _bn_relu(y + b_ref[...], g_ref[...], be_ref[...], N)


def _gru_body(H_ref, wih_ref, whh_ref, bih_ref, bhh_ref,
              w1_ref, b1_ref, g_ref, be_ref, w2_ref, b2_ref,
              out_ref, hstate):
    t = pl.program_id(0)

    @pl.when(t == 0)
    def _():
        hstate[...] = jnp.zeros((N, 128), jnp.float32)

    h = H_ref[0]
    gi = jnp.dot(h, wih_ref[...].T, preferred_element_type=jnp.float32) + bih_ref[...]
    gh = jnp.dot(hstate[...], whh_ref[...].T, preferred_element_type=jnp.float32) + bhh_ref[...]
    r = jax.nn.sigmoid(gi[:, :128] + gh[:, :128])
    z = jax.nn.sigmoid(gi[:, 128:256] + gh[:, 128:256])
    n_ = jnp.tanh(gi[:, 256:] + r * gh[:, 256:])
    hs = (1.0 - z) * n_ + z * hstate[...]
    hstate[...] = hs

    @pl.when(t == T - 1)
    def _():
        y = jnp.dot(hs, w1_ref[...].T, preferred_element_type=jnp.float32) + b1_ref[...]
        y = _bn_relu(y, g_ref[...], be_ref[...], N)
        out_ref[...] = jnp.dot(y, w2_ref[...].T, preferred_element_type=jnp.float32) + b2_ref[...]


def _full(shape):
    return pl.BlockSpec(shape, lambda t: tuple(0 for _ in shape))


def _per_t(shape):
    return pl.BlockSpec(shape, lambda t: (t,) + tuple(0 for _ in shape[1:]))


def kernel(x, edge_weight, params, edge_index):
    p = params
    f32 = jnp.float32

    npad = E_ALL - E_SL
    sl = jnp.arange(N, dtype=jnp.int32)
    row = jnp.concatenate(
        [edge_index[0], sl, (jnp.arange(npad, dtype=jnp.int32) * 37) % N])
    col = jnp.concatenate([edge_index[1], sl, jnp.zeros((npad,), jnp.int32)])
    ew = jnp.concatenate(
        [edge_weight.astype(f32), jnp.ones((N,), f32), jnp.zeros((npad,), f32)])
    ew_wide = jnp.broadcast_to(ew[:, None], (E_ALL, 16))
    zeros_sub = jnp.zeros((ROWS_PER_SUB, 128), f32)
    radj1 = row[None, :]
    radj8 = row[None, :] + (jnp.arange(T, dtype=jnp.int32) * N)[:, None]

    degp = _spmm_1(jnp.ones((N, 128), f32), radj1, col, ew_wide, zeros_sub)
    dp0 = degp[0, 0, :N, 0:1]
    dp1 = degp[1, 0, :N, 0:1]

    r2 = lambda v: v.reshape(1, -1)

    h0, a0 = pl.pallas_call(
        _pre_body,
        grid=(T,),
        in_specs=[
            _per_t((1, N, 128)),
            _full((N, 1)), _full((N, 1)),
            _full((256, 128)), _full((1, 256)), _full((1, 256)), _full((1, 256)),
            _full((128, 256)), _full((1, 128)), _full((1, 128)), _full((1, 128)),
            _full((128, 128)),
        ],
        out_specs=[_per_t((1, N, 128)), _per_t((1, N, 128))],
        out_shape=[jax.ShapeDtypeStruct((T, N, 128), f32),
                   jax.ShapeDtypeStruct((T, N, 128), f32)],
    )(x, dp0, dp1,
      p['pre_w1'], r2(p['pre_b1']), r2(p['pre_g1']), r2(p['pre_be1']),
      p['pre_w2'], r2(p['pre_b2']), r2(p['pre_g2']), r2(p['pre_be2']),
      p['gcn_w'][0])

    h, a = h0, a0
    for i in range(3):
        sp = _spmm_T(a.reshape(T * N, 128), radj8, col, ew_wide, zeros_sub)
        h = pl.pallas_call(
            _resid_body,
            grid=(T,),
            in_specs=[
                pl.BlockSpec((2, 1, NP, 128), lambda t: (0, t, 0, 0)),
                _per_t((1, N, 128)),
                _full((N, 1)), _full((N, 1)),
                _full((1, 128)), _full((1, 128)), _full((1, 128)),
            ],
            out_specs=_per_t((1, N, 128)),
            out_shape=jax.ShapeDtypeStruct((T, N, 128), f32),
        )(sp, h, dp0, dp1,
          r2(p['gcn_b'][i]), r2(p['gbn_g'][i]), r2(p['gbn_b'][i]))
        if i < 2:
            a = pl.pallas_call(
                _table_body,
                grid=(T,),
                in_specs=[_per_t((1, N, 128)), _full((N, 1)), _full((N, 1)),
                          _full((128, 128))],
                out_specs=_per_t((1, N, 128)),
                out_shape=jax.ShapeDtypeStruct((T, N, 128), f32),
            )(h, dp0, dp1, p['gcn_w'][i + 1])

    H = pl.pallas_call(
        _postmlp_body,
        grid=(T,),
        in_specs=[_per_t((1, N, 128)), _full((128, 128)),
                  _full((1, 128)), _full((1, 128)), _full((1, 128))],
        out_specs=_per_t((1, N, 128)),
        out_shape=jax.ShapeDtypeStruct((T, N, 128), f32),
    )(h, p['post_w'], r2(p['post_b']), r2(p['post_g']), r2(p['post_be']))

    out = pl.pallas_call(
        _gru_body,
        grid=(T,),
        in_specs=[
            _per_t((1, N, 128)),
            _full((384, 128)), _full((384, 128)), _full((1, 384)), _full((1, 384)),
            _full((256, 128)), _full((1, 256)), _full((1, 256)), _full((1, 256)),
            _full((128, 256)), _full((1, 128)),
        ],
        out_specs=_full((N, 128)),
        out_shape=jax.ShapeDtypeStruct((N, 128), f32),
        scratch_shapes=[pltpu.VMEM((N, 128), f32)],
    )(H, p['gru_wih'], p['gru_whh'], r2(p['gru_bih']), r2(p['gru_bhh']),
      p['cls_w1'], r2(p['cls_b1']), r2(p['cls_g']), r2(p['cls_be']),
      jnp.pad(p['cls_w2'], ((0, 128 - p['cls_w2'].shape[0]), (0, 0))),
      r2(jnp.pad(p['cls_b2'], (0, 128 - p['cls_b2'].shape[0]))))

    return out[:, :p['cls_b2'].shape[0]]


# speedup vs baseline: 6.5166x; 1.0035x over previous
import functools

import jax
import jax.numpy as jnp
from jax import lax
from jax.experimental import pallas as pl
from jax.experimental.pallas import tpu as pltpu
from jax.experimental.pallas import tpu_sc as plsc

N = 10000
T = 8
E = 320000
E_SL = E + N
NTILES = 32
SUB = 16
CHUNK = 64
NCH = 164
EPT = NCH * CHUNK
E_PAD = EPT * NTILES
E_ALL = E_PAD + CHUNK
NP = 10240
ROWS_PER_SUB = NP // SUB


def _make_spmm(nt):
    mesh = plsc.VectorSubcoreMesh(core_axis_name="c", subcore_axis_name="s")

    @functools.partial(
        pl.kernel,
        mesh=mesh,
        out_type=jax.ShapeDtypeStruct((2, nt, NP, 128), jnp.float32),
        scratch_types=[
            pltpu.VMEM((CHUNK,), jnp.int32),
            pltpu.VMEM((CHUNK,), jnp.int32),
            pltpu.VMEM((CHUNK,), jnp.int32),
            pltpu.VMEM((CHUNK,), jnp.int32),
            pltpu.VMEM((CHUNK,), jnp.int32),
            pltpu.VMEM((CHUNK,), jnp.int32),
            pltpu.VMEM((CHUNK, 16), jnp.float32),
            pltpu.VMEM((CHUNK, 16), jnp.float32),
            pltpu.VMEM((CHUNK, 128), jnp.float32),
            pltpu.VMEM((CHUNK, 128), jnp.float32),
            pltpu.VMEM_SHARED((NP, 128), jnp.float32),
            pltpu.SemaphoreType.DMA,
            pltpu.SemaphoreType.DMA,
            pltpu.SemaphoreType.DMA,
            pltpu.SemaphoreType.DMA,
            pltpu.SemaphoreType.DMA,
            pltpu.SemaphoreType.DMA,
        ],
    )
    def spmm(table_hbm, radj_hbm, col_hbm, ew_hbm, zeros_hbm, out_hbm,
             riA, riB, ciA, ciB, sciA, sciB, ewA, ewB, rowsA, rowsB, acc,
             sem_gA, sem_gB, sem_iA, sem_iB, sem_sA, sem_sB):
        c = lax.axis_index("c")
        s = lax.axis_index("s")
        wid = s * 2 + c
        ebase = wid * EPT

        def mul(rows_v, ew_v):
            def edge_body(e, carry2):
                bc = ew_v[e, :]
                for g in range(8):
                    rows_v[e, pl.ds(g * 16, 16)] = (
                        rows_v[e, pl.ds(g * 16, 16)] * bc)
                return carry2
            lax.fori_loop(0, CHUNK, edge_body, 0, unroll=2)

        def drain_idx(t, ri, ci, ew, sem):
            pltpu.make_async_copy(radj_hbm.at[t, pl.ds(0, CHUNK)], ri, sem).wait()
            pltpu.make_async_copy(col_hbm.at[pl.ds(0, CHUNK)], ci, sem).wait()
            pltpu.make_async_copy(ew_hbm.at[pl.ds(0, CHUNK)], ew, sem).wait()

        def step(t, j, cur, nxt, first, last):
            (ri0, ci0, sci0, ew0, rows0, sem_g0, sem_i0, sem_s0) = cur
            (ri1, ci1, sci1, ew1, rows1, sem_g1, sem_i1, sem_s1) = nxt
            if not first:
                pltpu.make_async_copy(rows1, acc.at[sci1], sem_s1).wait()
                drain_idx(t, ri1, ci1, ew1, sem_i1)
            if not last:
                pltpu.async_copy(table_hbm.at[ri1], rows1, sem_g1)
            pltpu.make_async_copy(table_hbm.at[ri0], rows0, sem_g0).wait()
            for g in range(CHUNK // 16):
                sci0[pl.ds(g * 16, 16)] = ci0[pl.ds(g * 16, 16)]
            if not last:
                base2 = ebase + (j + 2) * CHUNK
                pltpu.async_copy(radj_hbm.at[t, pl.ds(base2, CHUNK)], ri0, sem_i0)
                pltpu.async_copy(col_hbm.at[pl.ds(base2, CHUNK)], ci0, sem_i0)
            mul(rows0, ew0)
            pltpu.async_copy(rows0, acc.at[sci0], sem_s0, add=True)
            if not last:
                pltpu.async_copy(ew_hbm.at[pl.ds(base2, CHUNK)], ew0, sem_i0)

        bufA = (riA, ciA, sciA, ewA, rowsA, sem_gA, sem_iA, sem_sA)
        bufB = (riB, ciB, sciB, ewB, rowsB, sem_gB, sem_iB, sem_sB)

        def t_body(t, carry):
            pltpu.sync_copy(zeros_hbm, acc.at[pl.ds(s * ROWS_PER_SUB, ROWS_PER_SUB)])
            plsc.subcore_barrier()

            base0 = ebase
            pltpu.sync_copy(radj_hbm.at[t, pl.ds(base0, CHUNK)], riA)
            pltpu.sync_copy(col_hbm.at[pl.ds(base0, CHUNK)], ciA)
            pltpu.sync_copy(ew_hbm.at[pl.ds(base0, CHUNK)], ewA)
            pltpu.async_copy(table_hbm.at[riA], rowsA, sem_gA)
            base1 = ebase + CHUNK
            pltpu.sync_copy(radj_hbm.at[t, pl.ds(base1, CHUNK)], riB)
            pltpu.sync_copy(col_hbm.at[pl.ds(base1, CHUNK)], ciB)
            pltpu.sync_copy(ew_hbm.at[pl.ds(base1, CHUNK)], ewB)

            step(t, 0, bufA, bufB, True, False)
            step(t, 1, bufB, bufA, False, False)

            def pair_body(k, carry2):
                step(t, 2 * k, bufA, bufB, False, False)
                step(t, 2 * k + 1, bufB, bufA, False, False)
                return carry2
            lax.fori_loop(1, NCH // 2 - 1, pair_body, 0)
            step(t, NCH - 2, bufA, bufB, False, False)
            step(t, NCH - 1, bufB, bufA, False, True)
            pltpu.make_async_copy(rowsB, acc.at[sciB], sem_sB).wait()

            plsc.subcore_barrier()
            pltpu.sync_copy(
                acc.at[pl.ds(s * ROWS_PER_SUB, ROWS_PER_SUB)],
                out_hbm.at[c, t, pl.ds(s * ROWS_PER_SUB, ROWS_PER_SUB)])
            plsc.subcore_barrier()
            return carry

        lax.fori_loop(0, nt, t_body, 0)

    return spmm


_spmm_T = _make_spmm(T)
_spmm_1 = _make_spmm(1)


def _bn_relu(y, g, b, n_rows):
    mu = jnp.sum(y, axis=0, keepdims=True) / n_rows
    var = jnp.sum((y - mu) ** 2, axis=0, keepdims=True) / n_rows
    return jnp.maximum(g * (y - mu) / jnp.sqrt(var + 1e-5) + b, 0.0)


def _dis_from_parts(p0, p1):
    return lax.rsqrt(p0 + p1)


def _pre_body(x_ref, p0_ref, p1_ref, w1_ref, b1_ref, g1_ref, be1_ref,
              w2_ref, b2_ref, g2_ref, be2_ref, gw_ref,
              h_ref, a_ref):
    x = x_ref[0]
    h = jnp.dot(x, w1_ref[...].T, preferred_element_type=jnp.float32) + b1_ref[...]
    h = _bn_relu(h, g1_ref[...], be1_ref[...], N)
    h = jnp.dot(h, w2_ref[...].T, preferred_element_type=jnp.float32) + b2_ref[...]
    h = _bn_relu(h, g2_ref[...], be2_ref[...], N)
    h_ref[0] = h
    dis = _dis_from_parts(p0_ref[...], p1_ref[...])
    a_ref[0] = dis * jnp.dot(h, gw_ref[...].T, preferred_element_type=jnp.float32)


def _resid_body(sp_ref, h_ref, p0_ref, p1_ref, gb_ref, g_ref, be_ref,
                h_out_ref):
    dis = _dis_from_parts(p0_ref[...], p1_ref[...])
    srow = sp_ref[0, 0, :N] + sp_ref[1, 0, :N]
    hn = dis * srow + gb_ref[...]
    hn = _bn_relu(hn, g_ref[...], be_ref[...], N)
    h_out_ref[0] = h_ref[0] + hn


def _table_body(h_ref, p0_ref, p1_ref, w_ref, a_ref):
    dis = _dis_from_parts(p0_ref[...], p1_ref[...])
    a_ref[0] = dis * jnp.dot(h_ref[0], w_ref[...].T,
                             preferred_element_type=jnp.float32)


def _postmlp_body(h_ref, w_ref, b_ref, g_ref, be_ref, out_ref):
    y = jnp.dot(h_ref[0], w_ref[...].T, preferred_element_type=jnp.float32)
    out_ref[0] = _bn_relu(y + b_ref[...], g_ref[...], be_ref[...], N)


def _gru_body(H_ref, wih_ref, whh_ref, bih_ref, bhh_ref,
              w1_ref, b1_ref, g_ref, be_ref, w2_ref, b2_ref,
              out_ref, hstate):
    t = pl.program_id(0)

    @pl.when(t == 0)
    def _():
        hstate[...] = jnp.zeros((N, 128), jnp.float32)

    h = H_ref[0]
    gi = jnp.dot(h, wih_ref[...].T, preferred_element_type=jnp.float32) + bih_ref[...]
    gh = jnp.dot(hstate[...], whh_ref[...].T, preferred_element_type=jnp.float32) + bhh_ref[...]
    r = jax.nn.sigmoid(gi[:, :128] + gh[:, :128])
    z = jax.nn.sigmoid(gi[:, 128:256] + gh[:, 128:256])
    n_ = jnp.tanh(gi[:, 256:] + r * gh[:, 256:])
    hs = (1.0 - z) * n_ + z * hstate[...]
    hstate[...] = hs

    @pl.when(t == T - 1)
    def _():
        y = jnp.dot(hs, w1_ref[...].T, preferred_element_type=jnp.float32) + b1_ref[...]
        y = _bn_relu(y, g_ref[...], be_ref[...], N)
        out_ref[...] = jnp.dot(y, w2_ref[...].T, preferred_element_type=jnp.float32) + b2_ref[...]


def _full(shape):
    return pl.BlockSpec(shape, lambda t: tuple(0 for _ in shape))


def _per_t(shape):
    return pl.BlockSpec(shape, lambda t: (t,) + tuple(0 for _ in shape[1:]))


def kernel(x, edge_weight, params, edge_index):
    p = params
    f32 = jnp.float32

    npad = E_ALL - E_SL
    sl = jnp.arange(N, dtype=jnp.int32)
    row = jnp.concatenate(
        [edge_index[0], sl, (jnp.arange(npad, dtype=jnp.int32) * 37) % N])
    col = jnp.concatenate([edge_index[1], sl, jnp.zeros((npad,), jnp.int32)])
    ew = jnp.concatenate(
        [edge_weight.astype(f32), jnp.ones((N,), f32), jnp.zeros((npad,), f32)])
    ew_wide = jnp.broadcast_to(ew[:, None], (E_ALL, 16))
    zeros_sub = jnp.zeros((ROWS_PER_SUB, 128), f32)
    radj1 = row[None, :]
    radj8 = row[None, :] + (jnp.arange(T, dtype=jnp.int32) * N)[:, None]

    degp = _spmm_1(jnp.ones((N, 128), f32), radj1, col, ew_wide, zeros_sub)
    dp0 = degp[0, 0, :N, 0:1]
    dp1 = degp[1, 0, :N, 0:1]

    r2 = lambda v: v.reshape(1, -1)

    h0, a0 = pl.pallas_call(
        _pre_body,
        grid=(T,),
        in_specs=[
            _per_t((1, N, 128)),
            _full((N, 1)), _full((N, 1)),
            _full((256, 128)), _full((1, 256)), _full((1, 256)), _full((1, 256)),
            _full((128, 256)), _full((1, 128)), _full((1, 128)), _full((1, 128)),
            _full((128, 128)),
        ],
        out_specs=[_per_t((1, N, 128)), _per_t((1, N, 128))],
        out_shape=[jax.ShapeDtypeStruct((T, N, 128), f32),
                   jax.ShapeDtypeStruct((T, N, 128), f32)],
    )(x, dp0, dp1,
      p['pre_w1'], r2(p['pre_b1']), r2(p['pre_g1']), r2(p['pre_be1']),
      p['pre_w2'], r2(p['pre_b2']), r2(p['pre_g2']), r2(p['pre_be2']),
      p['gcn_w'][0])

    h, a = h0, a0
    for i in range(3):
        sp = _spmm_T(a.reshape(T * N, 128), radj8, col, ew_wide, zeros_sub)
        h = pl.pallas_call(
            _resid_body,
            grid=(T,),
            in_specs=[
                pl.BlockSpec((2, 1, NP, 128), lambda t: (0, t, 0, 0)),
                _per_t((1, N, 128)),
                _full((N, 1)), _full((N, 1)),
                _full((1, 128)), _full((1, 128)), _full((1, 128)),
            ],
            out_specs=_per_t((1, N, 128)),
            out_shape=jax.ShapeDtypeStruct((T, N, 128), f32),
        )(sp, h, dp0, dp1,
          r2(p['gcn_b'][i]), r2(p['gbn_g'][i]), r2(p['gbn_b'][i]))
        if i < 2:
            a = pl.pallas_call(
                _table_body,
                grid=(T,),
                in_specs=[_per_t((1, N, 128)), _full((N, 1)), _full((N, 1)),
                          _full((128, 128))],
                out_specs=_per_t((1, N, 128)),
                out_shape=jax.ShapeDtypeStruct((T, N, 128), f32),
            )(h, dp0, dp1, p['gcn_w'][i + 1])

    H = pl.pallas_call(
        _postmlp_body,
        grid=(T,),
        in_specs=[_per_t((1, N, 128)), _full((128, 128)),
                  _full((1, 128)), _full((1, 128)), _full((1, 128))],
        out_specs=_per_t((1, N, 128)),
        out_shape=jax.ShapeDtypeStruct((T, N, 128), f32),
    )(h, p['post_w'], r2(p['post_b']), r2(p['post_g']), r2(p['post_be']))

    out = pl.pallas_call(
        _gru_body,
        grid=(T,),
        in_specs=[
            _per_t((1, N, 128)),
            _full((384, 128)), _full((384, 128)), _full((1, 384)), _full((1, 384)),
            _full((256, 128)), _full((1, 256)), _full((1, 256)), _full((1, 256)),
            _full((128, 256)), _full((1, 128)),
        ],
        out_specs=_full((N, 128)),
        out_shape=jax.ShapeDtypeStruct((N, 128), f32),
        scratch_shapes=[pltpu.VMEM((N, 128), f32)],
    )(H, p['gru_wih'], p['gru_whh'], r2(p['gru_bih']), r2(p['gru_bhh']),
      p['cls_w1'], r2(p['cls_b1']), r2(p['cls_g']), r2(p['cls_be']),
      jnp.pad(p['cls_w2'], ((0, 128 - p['cls_w2'].shape[0]), (0, 0))),
      r2(jnp.pad(p['cls_b2'], (0, 128 - p['cls_b2'].shape[0]))))

    return out[:, :p['cls_b2'].shape[0]]


# speedup vs baseline: 8.2719x; 1.2694x over previous
import functools

import jax
import jax.numpy as jnp
from jax import lax
from jax.experimental import pallas as pl
from jax.experimental.pallas import tpu as pltpu
from jax.experimental.pallas import tpu_sc as plsc

N = 10000
T = 8
E = 320000
E_SL = E + N
NTILES = 32
SUB = 16
CHUNK = 64
NCH = 164
EPT = NCH * CHUNK
E_PAD = EPT * NTILES
E_ALL = E_PAD + CHUNK
NP = 10240
ROWS_PER_SUB = NP // SUB


EW_OFF = 2 * CHUNK
EROW = EW_OFF + 16 * CHUNK


def _make_spmm(nt):
    mesh = plsc.VectorSubcoreMesh(core_axis_name="c", subcore_axis_name="s")

    @functools.partial(
        pl.kernel,
        mesh=mesh,
        out_type=jax.ShapeDtypeStruct((2, nt, NP, 128), jnp.float32),
        scratch_types=[
            pltpu.VMEM((EROW,), jnp.int32),
            pltpu.VMEM((EROW,), jnp.int32),
            pltpu.VMEM((CHUNK,), jnp.int32),
            pltpu.VMEM((CHUNK,), jnp.int32),
            pltpu.VMEM((CHUNK,), jnp.int32),
            pltpu.VMEM((CHUNK,), jnp.int32),
            pltpu.VMEM((CHUNK, 128), jnp.float32),
            pltpu.VMEM((CHUNK, 128), jnp.float32),
            pltpu.VMEM_SHARED((NP, 128), jnp.float32),
            pltpu.SemaphoreType.DMA,
            pltpu.SemaphoreType.DMA,
            pltpu.SemaphoreType.DMA,
            pltpu.SemaphoreType.DMA,
            pltpu.SemaphoreType.DMA,
            pltpu.SemaphoreType.DMA,
        ],
    )
    def spmm(table_hbm, edata_hbm, zeros_hbm, out_hbm,
             ebA, ebB, riA, riB, sciA, sciB, rowsA, rowsB, acc,
             sem_gA, sem_gB, sem_iA, sem_iB, sem_sA, sem_sB):
        c = lax.axis_index("c")
        s = lax.axis_index("s")
        wid = s * 2 + c
        cbase = wid * NCH

        def mul(rows_v, eb):
            def edge_body(e, carry2):
                bc = lax.bitcast_convert_type(
                    eb[pl.ds(EW_OFF + e * 16, 16)], jnp.float32)
                for g in range(8):
                    rows_v[e, pl.ds(g * 16, 16)] = (
                        rows_v[e, pl.ds(g * 16, 16)] * bc)
                return carry2
            lax.fori_loop(0, CHUNK, edge_body, 0, unroll=2)

        def adjust(t, eb, ri):
            off = jnp.full((16,), t * N, jnp.int32)
            for g in range(CHUNK // 16):
                ri[pl.ds(g * 16, 16)] = eb[pl.ds(g * 16, 16)] + off

        def step(t, j, cur, nxt, first, last):
            (eb0, ri0, sci0, rows0, sem_g0, sem_i0, sem_s0) = cur
            (eb1, ri1, sci1, rows1, sem_g1, sem_i1, sem_s1) = nxt
            if not first:
                pltpu.make_async_copy(rows1, acc.at[sci1], sem_s1).wait()
                pltpu.make_async_copy(edata_hbm.at[0], eb1, sem_i1).wait()
            if not last:
                adjust(t, eb1, ri1)
                pltpu.async_copy(table_hbm.at[ri1], rows1, sem_g1)
            pltpu.make_async_copy(table_hbm.at[ri0], rows0, sem_g0).wait()
            for g in range(CHUNK // 16):
                sci0[pl.ds(g * 16, 16)] = eb0[pl.ds(CHUNK + g * 16, 16)]
            mul(rows0, eb0)
            pltpu.async_copy(rows0, acc.at[sci0], sem_s0, add=True)
            if not last:
                pltpu.async_copy(edata_hbm.at[cbase + j + 2], eb0, sem_i0)

        bufA = (ebA, riA, sciA, rowsA, sem_gA, sem_iA, sem_sA)
        bufB = (ebB, riB, sciB, rowsB, sem_gB, sem_iB, sem_sB)

        def t_body(t, carry):
            pltpu.sync_copy(zeros_hbm, acc.at[pl.ds(s * ROWS_PER_SUB, ROWS_PER_SUB)])
            plsc.subcore_barrier()

            pltpu.sync_copy(edata_hbm.at[cbase], ebA)
            adjust(t, ebA, riA)
            pltpu.async_copy(table_hbm.at[riA], rowsA, sem_gA)
            pltpu.sync_copy(edata_hbm.at[cbase + 1], ebB)

            step(t, 0, bufA, bufB, True, False)
            step(t, 1, bufB, bufA, False, False)

            def pair_body(k, carry2):
                step(t, 2 * k, bufA, bufB, False, False)
                step(t, 2 * k + 1, bufB, bufA, False, False)
                return carry2
            lax.fori_loop(1, NCH // 2 - 1, pair_body, 0)
            step(t, NCH - 2, bufA, bufB, False, False)
            step(t, NCH - 1, bufB, bufA, False, True)
            pltpu.make_async_copy(rowsB, acc.at[sciB], sem_sB).wait()

            plsc.subcore_barrier()
            pltpu.sync_copy(
                acc.at[pl.ds(s * ROWS_PER_SUB, ROWS_PER_SUB)],
                out_hbm.at[c, t, pl.ds(s * ROWS_PER_SUB, ROWS_PER_SUB)])
            plsc.subcore_barrier()
            return carry

        lax.fori_loop(0, nt, t_body, 0)

    return spmm


_spmm_T = _make_spmm(T)
_spmm_1 = _make_spmm(1)


def _bn_relu(y, g, b, n_rows):
    mu = jnp.sum(y, axis=0, keepdims=True) / n_rows
    var = jnp.sum((y - mu) ** 2, axis=0, keepdims=True) / n_rows
    return jnp.maximum(g * (y - mu) / jnp.sqrt(var + 1e-5) + b, 0.0)


def _dis_from_parts(p0, p1):
    return lax.rsqrt(p0 + p1)


def _pre_body(x_ref, p0_ref, p1_ref, w1_ref, b1_ref, g1_ref, be1_ref,
              w2_ref, b2_ref, g2_ref, be2_ref, gw_ref,
              h_ref, a_ref):
    x = x_ref[0]
    h = jnp.dot(x, w1_ref[...].T, preferred_element_type=jnp.float32) + b1_ref[...]
    h = _bn_relu(h, g1_ref[...], be1_ref[...], N)
    h = jnp.dot(h, w2_ref[...].T, preferred_element_type=jnp.float32) + b2_ref[...]
    h = _bn_relu(h, g2_ref[...], be2_ref[...], N)
    h_ref[0] = h
    dis = _dis_from_parts(p0_ref[...], p1_ref[...])
    a_ref[0] = dis * jnp.dot(h, gw_ref[...].T, preferred_element_type=jnp.float32)


def _resid_body(sp_ref, h_ref, p0_ref, p1_ref, gb_ref, g_ref, be_ref,
                h_out_ref):
    dis = _dis_from_parts(p0_ref[...], p1_ref[...])
    srow = sp_ref[0, 0, :N] + sp_ref[1, 0, :N]
    hn = dis * srow + gb_ref[...]
    hn = _bn_relu(hn, g_ref[...], be_ref[...], N)
    h_out_ref[0] = h_ref[0] + hn


def _table_body(h_ref, p0_ref, p1_ref, w_ref, a_ref):
    dis = _dis_from_parts(p0_ref[...], p1_ref[...])
    a_ref[0] = dis * jnp.dot(h_ref[0], w_ref[...].T,
                             preferred_element_type=jnp.float32)


def _postmlp_body(h_ref, w_ref, b_ref, g_ref, be_ref, out_ref):
    y = jnp.dot(h_ref[0], w_ref[...].T, preferred_element_type=jnp.float32)
    out_ref[0] = _bn_relu(y + b_ref[...], g_ref[...], be_ref[...], N)


def _gru_body(H_ref, wih_ref, whh_ref, bih_ref, bhh_ref,
              w1_ref, b1_ref, g_ref, be_ref, w2_ref, b2_ref,
              out_ref, hstate):
    t = pl.program_id(0)

    @pl.when(t == 0)
    def _():
        hstate[...] = jnp.zeros((N, 128), jnp.float32)

    h = H_ref[0]
    gi = jnp.dot(h, wih_ref[...].T, preferred_element_type=jnp.float32) + bih_ref[...]
    gh = jnp.dot(hstate[...], whh_ref[...].T, preferred_element_type=jnp.float32) + bhh_ref[...]
    r = jax.nn.sigmoid(gi[:, :128] + gh[:, :128])
    z = jax.nn.sigmoid(gi[:, 128:256] + gh[:, 128:256])
    n_ = jnp.tanh(gi[:, 256:] + r * gh[:, 256:])
    hs = (1.0 - z) * n_ + z * hstate[...]
    hstate[...] = hs

    @pl.when(t == T - 1)
    def _():
        y = jnp.dot(hs, w1_ref[...].T, preferred_element_type=jnp.float32) + b1_ref[...]
        y = _bn_relu(y, g_ref[...], be_ref[...], N)
        out_ref[...] = jnp.dot(y, w2_ref[...].T, preferred_element_type=jnp.float32) + b2_ref[...]


def _full(shape):
    return pl.BlockSpec(shape, lambda t: tuple(0 for _ in shape))


def _per_t(shape):
    return pl.BlockSpec(shape, lambda t: (t,) + tuple(0 for _ in shape[1:]))


def kernel(x, edge_weight, params, edge_index):
    p = params
    f32 = jnp.float32

    npad = E_ALL - E_SL
    sl = jnp.arange(N, dtype=jnp.int32)
    row = jnp.concatenate(
        [edge_index[0], sl, (jnp.arange(npad, dtype=jnp.int32) * 37) % N])
    col = jnp.concatenate([edge_index[1], sl, jnp.zeros((npad,), jnp.int32)])
    ew = jnp.concatenate(
        [edge_weight.astype(f32), jnp.ones((N,), f32), jnp.zeros((npad,), f32)])
    ew_wide = jnp.broadcast_to(ew[:, None], (E_ALL, 16))
    zeros_sub = jnp.zeros((ROWS_PER_SUB, 128), f32)
    ncht = E_ALL // CHUNK
    edata = jnp.concatenate(
        [row.reshape(ncht, CHUNK), col.reshape(ncht, CHUNK),
         jax.lax.bitcast_convert_type(ew_wide, jnp.int32).reshape(ncht, 16 * CHUNK)],
        axis=1)

    degp = _spmm_1(jnp.ones((N, 128), f32), edata, zeros_sub)
    dp0 = degp[0, 0, :N, 0:1]
    dp1 = degp[1, 0, :N, 0:1]

    r2 = lambda v: v.reshape(1, -1)

    h0, a0 = pl.pallas_call(
        _pre_body,
        grid=(T,),
        in_specs=[
            _per_t((1, N, 128)),
            _full((N, 1)), _full((N, 1)),
            _full((256, 128)), _full((1, 256)), _full((1, 256)), _full((1, 256)),
            _full((128, 256)), _full((1, 128)), _full((1, 128)), _full((1, 128)),
            _full((128, 128)),
        ],
        out_specs=[_per_t((1, N, 128)), _per_t((1, N, 128))],
        out_shape=[jax.ShapeDtypeStruct((T, N, 128), f32),
                   jax.ShapeDtypeStruct((T, N, 128), f32)],
    )(x, dp0, dp1,
      p['pre_w1'], r2(p['pre_b1']), r2(p['pre_g1']), r2(p['pre_be1']),
      p['pre_w2'], r2(p['pre_b2']), r2(p['pre_g2']), r2(p['pre_be2']),
      p['gcn_w'][0])

    h, a = h0, a0
    for i in range(3):
        sp = _spmm_T(a.reshape(T * N, 128), edata, zeros_sub)
        h = pl.pallas_call(
            _resid_body,
            grid=(T,),
            in_specs=[
                pl.BlockSpec((2, 1, NP, 128), lambda t: (0, t, 0, 0)),
                _per_t((1, N, 128)),
                _full((N, 1)), _full((N, 1)),
                _full((1, 128)), _full((1, 128)), _full((1, 128)),
            ],
            out_specs=_per_t((1, N, 128)),
            out_shape=jax.ShapeDtypeStruct((T, N, 128), f32),
        )(sp, h, dp0, dp1,
          r2(p['gcn_b'][i]), r2(p['gbn_g'][i]), r2(p['gbn_b'][i]))
        if i < 2:
            a = pl.pallas_call(
                _table_body,
                grid=(T,),
                in_specs=[_per_t((1, N, 128)), _full((N, 1)), _full((N, 1)),
                          _full((128, 128))],
                out_specs=_per_t((1, N, 128)),
                out_shape=jax.ShapeDtypeStruct((T, N, 128), f32),
            )(h, dp0, dp1, p['gcn_w'][i + 1])

    H = pl.pallas_call(
        _postmlp_body,
        grid=(T,),
        in_specs=[_per_t((1, N, 128)), _full((128, 128)),
                  _full((1, 128)), _full((1, 128)), _full((1, 128))],
        out_specs=_per_t((1, N, 128)),
        out_shape=jax.ShapeDtypeStruct((T, N, 128), f32),
    )(h, p['post_w'], r2(p['post_b']), r2(p['post_g']), r2(p['post_be']))

    out = pl.pallas_call(
        _gru_body,
        grid=(T,),
        in_specs=[
            _per_t((1, N, 128)),
            _full((384, 128)), _full((384, 128)), _full((1, 384)), _full((1, 384)),
            _full((256, 128)), _full((1, 256)), _full((1, 256)), _full((1, 256)),
            _full((128, 256)), _full((1, 128)),
        ],
        out_specs=_full((N, 128)),
        out_shape=jax.ShapeDtypeStruct((N, 128), f32),
        scratch_shapes=[pltpu.VMEM((N, 128), f32)],
    )(H, p['gru_wih'], p['gru_whh'], r2(p['gru_bih']), r2(p['gru_bhh']),
      p['cls_w1'], r2(p['cls_b1']), r2(p['cls_g']), r2(p['cls_be']),
      jnp.pad(p['cls_w2'], ((0, 128 - p['cls_w2'].shape[0]), (0, 0))),
      r2(jnp.pad(p['cls_b2'], (0, 128 - p['cls_b2'].shape[0]))))

    return out[:, :p['cls_b2'].shape[0]]


# speedup vs baseline: 10.1832x; 1.2311x over previous
import functools

import jax
import jax.numpy as jnp
from jax import lax
from jax.experimental import pallas as pl
from jax.experimental.pallas import tpu as pltpu
from jax.experimental.pallas import tpu_sc as plsc

N = 10000
T = 8
E = 320000
E_SL = E + N
NTILES = 32
SUB = 16
CHUNK = 128
NCH = 82
EPT = NCH * CHUNK
E_PAD = EPT * NTILES
E_ALL = E_PAD + CHUNK
NP = 10240
ROWS_PER_SUB = NP // SUB


EW_OFF = 2 * CHUNK
EROW = EW_OFF + 16 * CHUNK


def _make_spmm(nt):
    mesh = plsc.VectorSubcoreMesh(core_axis_name="c", subcore_axis_name="s")

    @functools.partial(
        pl.kernel,
        mesh=mesh,
        out_type=jax.ShapeDtypeStruct((2, nt, NP, 128), jnp.float32),
        scratch_types=[
            pltpu.VMEM((EROW,), jnp.int32),
            pltpu.VMEM((EROW,), jnp.int32),
            pltpu.VMEM((CHUNK,), jnp.int32),
            pltpu.VMEM((CHUNK,), jnp.int32),
            pltpu.VMEM((CHUNK,), jnp.int32),
            pltpu.VMEM((CHUNK,), jnp.int32),
            pltpu.VMEM((CHUNK, 128), jnp.float32),
            pltpu.VMEM((CHUNK, 128), jnp.float32),
            pltpu.VMEM_SHARED((NP, 128), jnp.float32),
            pltpu.SemaphoreType.DMA,
            pltpu.SemaphoreType.DMA,
            pltpu.SemaphoreType.DMA,
            pltpu.SemaphoreType.DMA,
            pltpu.SemaphoreType.DMA,
            pltpu.SemaphoreType.DMA,
        ],
    )
    def spmm(table_hbm, edata_hbm, zeros_hbm, out_hbm,
             ebA, ebB, riA, riB, sciA, sciB, rowsA, rowsB, acc,
             sem_gA, sem_gB, sem_iA, sem_iB, sem_sA, sem_sB):
        c = lax.axis_index("c")
        s = lax.axis_index("s")
        wid = s * 2 + c
        cbase = wid * NCH

        def mul(rows_v, eb):
            def edge_body(e, carry2):
                bc = lax.bitcast_convert_type(
                    eb[pl.ds(EW_OFF + e * 16, 16)], jnp.float32)
                for g in range(8):
                    rows_v[e, pl.ds(g * 16, 16)] = (
                        rows_v[e, pl.ds(g * 16, 16)] * bc)
                return carry2
            lax.fori_loop(0, CHUNK, edge_body, 0, unroll=2)

        def adjust(t, eb, ri):
            off = jnp.full((16,), t * N, jnp.int32)
            for g in range(CHUNK // 16):
                ri[pl.ds(g * 16, 16)] = eb[pl.ds(g * 16, 16)] + off

        def step(t, j, cur, nxt, first, last):
            (eb0, ri0, sci0, rows0, sem_g0, sem_i0, sem_s0) = cur
            (eb1, ri1, sci1, rows1, sem_g1, sem_i1, sem_s1) = nxt
            if not first:
                pltpu.make_async_copy(rows1, acc.at[sci1], sem_s1).wait()
                pltpu.make_async_copy(edata_hbm.at[0], eb1, sem_i1).wait()
            if not last:
                adjust(t, eb1, ri1)
                pltpu.async_copy(table_hbm.at[ri1], rows1, sem_g1)
            pltpu.make_async_copy(table_hbm.at[ri0], rows0, sem_g0).wait()
            for g in range(CHUNK // 16):
                sci0[pl.ds(g * 16, 16)] = eb0[pl.ds(CHUNK + g * 16, 16)]
            mul(rows0, eb0)
            pltpu.async_copy(rows0, acc.at[sci0], sem_s0, add=True)
            if not last:
                pltpu.async_copy(edata_hbm.at[cbase + j + 2], eb0, sem_i0)

        bufA = (ebA, riA, sciA, rowsA, sem_gA, sem_iA, sem_sA)
        bufB = (ebB, riB, sciB, rowsB, sem_gB, sem_iB, sem_sB)

        def t_body(t, carry):
            pltpu.sync_copy(zeros_hbm, acc.at[pl.ds(s * ROWS_PER_SUB, ROWS_PER_SUB)])
            plsc.subcore_barrier()

            pltpu.sync_copy(edata_hbm.at[cbase], ebA)
            adjust(t, ebA, riA)
            pltpu.async_copy(table_hbm.at[riA], rowsA, sem_gA)
            pltpu.sync_copy(edata_hbm.at[cbase + 1], ebB)

            step(t, 0, bufA, bufB, True, False)
            step(t, 1, bufB, bufA, False, False)

            def pair_body(k, carry2):
                step(t, 2 * k, bufA, bufB, False, False)
                step(t, 2 * k + 1, bufB, bufA, False, False)
                return carry2
            lax.fori_loop(1, NCH // 2 - 1, pair_body, 0)
            step(t, NCH - 2, bufA, bufB, False, False)
            step(t, NCH - 1, bufB, bufA, False, True)
            pltpu.make_async_copy(rowsB, acc.at[sciB], sem_sB).wait()

            plsc.subcore_barrier()
            pltpu.sync_copy(
                acc.at[pl.ds(s * ROWS_PER_SUB, ROWS_PER_SUB)],
                out_hbm.at[c, t, pl.ds(s * ROWS_PER_SUB, ROWS_PER_SUB)])
            plsc.subcore_barrier()
            return carry

        lax.fori_loop(0, nt, t_body, 0)

    return spmm


_spmm_T = _make_spmm(T)
_spmm_1 = _make_spmm(1)


def _bn_relu(y, g, b, n_rows):
    mu = jnp.sum(y, axis=0, keepdims=True) / n_rows
    var = jnp.sum((y - mu) ** 2, axis=0, keepdims=True) / n_rows
    return jnp.maximum(g * (y - mu) / jnp.sqrt(var + 1e-5) + b, 0.0)


def _dis_from_parts(p0, p1):
    return lax.rsqrt(p0 + p1)


def _pre_body(x_ref, p0_ref, p1_ref, w1_ref, b1_ref, g1_ref, be1_ref,
              w2_ref, b2_ref, g2_ref, be2_ref, gw_ref,
              h_ref, a_ref):
    x = x_ref[0]
    h = jnp.dot(x, w1_ref[...].T, preferred_element_type=jnp.float32) + b1_ref[...]
    h = _bn_relu(h, g1_ref[...], be1_ref[...], N)
    h = jnp.dot(h, w2_ref[...].T, preferred_element_type=jnp.float32) + b2_ref[...]
    h = _bn_relu(h, g2_ref[...], be2_ref[...], N)
    h_ref[0] = h
    dis = _dis_from_parts(p0_ref[...], p1_ref[...])
    a_ref[0] = dis * jnp.dot(h, gw_ref[...].T, preferred_element_type=jnp.float32)


def _resid_body(sp_ref, h_ref, p0_ref, p1_ref, gb_ref, g_ref, be_ref,
                h_out_ref):
    dis = _dis_from_parts(p0_ref[...], p1_ref[...])
    srow = sp_ref[0, 0, :N] + sp_ref[1, 0, :N]
    hn = dis * srow + gb_ref[...]
    hn = _bn_relu(hn, g_ref[...], be_ref[...], N)
    h_out_ref[0] = h_ref[0] + hn


def _table_body(h_ref, p0_ref, p1_ref, w_ref, a_ref):
    dis = _dis_from_parts(p0_ref[...], p1_ref[...])
    a_ref[0] = dis * jnp.dot(h_ref[0], w_ref[...].T,
                             preferred_element_type=jnp.float32)


def _postmlp_body(h_ref, w_ref, b_ref, g_ref, be_ref, out_ref):
    y = jnp.dot(h_ref[0], w_ref[...].T, preferred_element_type=jnp.float32)
    out_ref[0] = _bn_relu(y + b_ref[...], g_ref[...], be_ref[...], N)


def _gru_body(H_ref, wih_ref, whh_ref, bih_ref, bhh_ref,
              w1_ref, b1_ref, g_ref, be_ref, w2_ref, b2_ref,
              out_ref, hstate):
    t = pl.program_id(0)

    @pl.when(t == 0)
    def _():
        hstate[...] = jnp.zeros((N, 128), jnp.float32)

    h = H_ref[0]
    gi = jnp.dot(h, wih_ref[...].T, preferred_element_type=jnp.float32) + bih_ref[...]
    gh = jnp.dot(hstate[...], whh_ref[...].T, preferred_element_type=jnp.float32) + bhh_ref[...]
    r = jax.nn.sigmoid(gi[:, :128] + gh[:, :128])
    z = jax.nn.sigmoid(gi[:, 128:256] + gh[:, 128:256])
    n_ = jnp.tanh(gi[:, 256:] + r * gh[:, 256:])
    hs = (1.0 - z) * n_ + z * hstate[...]
    hstate[...] = hs

    @pl.when(t == T - 1)
    def _():
        y = jnp.dot(hs, w1_ref[...].T, preferred_element_type=jnp.float32) + b1_ref[...]
        y = _bn_relu(y, g_ref[...], be_ref[...], N)
        out_ref[...] = jnp.dot(y, w2_ref[...].T, preferred_element_type=jnp.float32) + b2_ref[...]


def _full(shape):
    return pl.BlockSpec(shape, lambda t: tuple(0 for _ in shape))


def _per_t(shape):
    return pl.BlockSpec(shape, lambda t: (t,) + tuple(0 for _ in shape[1:]))


def kernel(x, edge_weight, params, edge_index):
    p = params
    f32 = jnp.float32

    npad = E_ALL - E_SL
    sl = jnp.arange(N, dtype=jnp.int32)
    row = jnp.concatenate(
        [edge_index[0], sl, (jnp.arange(npad, dtype=jnp.int32) * 37) % N])
    col = jnp.concatenate([edge_index[1], sl, jnp.zeros((npad,), jnp.int32)])
    ew = jnp.concatenate(
        [edge_weight.astype(f32), jnp.ones((N,), f32), jnp.zeros((npad,), f32)])
    ew_wide = jnp.broadcast_to(ew[:, None], (E_ALL, 16))
    zeros_sub = jnp.zeros((ROWS_PER_SUB, 128), f32)
    ncht = E_ALL // CHUNK
    edata = jnp.concatenate(
        [row.reshape(ncht, CHUNK), col.reshape(ncht, CHUNK),
         jax.lax.bitcast_convert_type(ew_wide, jnp.int32).reshape(ncht, 16 * CHUNK)],
        axis=1)

    degp = _spmm_1(jnp.ones((N, 128), f32), edata, zeros_sub)
    dp0 = degp[0, 0, :N, 0:1]
    dp1 = degp[1, 0, :N, 0:1]

    r2 = lambda v: v.reshape(1, -1)

    h0, a0 = pl.pallas_call(
        _pre_body,
        grid=(T,),
        in_specs=[
            _per_t((1, N, 128)),
            _full((N, 1)), _full((N, 1)),
            _full((256, 128)), _full((1, 256)), _full((1, 256)), _full((1, 256)),
            _full((128, 256)), _full((1, 128)), _full((1, 128)), _full((1, 128)),
            _full((128, 128)),
        ],
        out_specs=[_per_t((1, N, 128)), _per_t((1, N, 128))],
        out_shape=[jax.ShapeDtypeStruct((T, N, 128), f32),
                   jax.ShapeDtypeStruct((T, N, 128), f32)],
    )(x, dp0, dp1,
      p['pre_w1'], r2(p['pre_b1']), r2(p['pre_g1']), r2(p['pre_be1']),
      p['pre_w2'], r2(p['pre_b2']), r2(p['pre_g2']), r2(p['pre_be2']),
      p['gcn_w'][0])

    h, a = h0, a0
    for i in range(3):
        sp = _spmm_T(a.reshape(T * N, 128), edata, zeros_sub)
        h = pl.pallas_call(
            _resid_body,
            grid=(T,),
            in_specs=[
                pl.BlockSpec((2, 1, NP, 128), lambda t: (0, t, 0, 0)),
                _per_t((1, N, 128)),
                _full((N, 1)), _full((N, 1)),
                _full((1, 128)), _full((1, 128)), _full((1, 128)),
            ],
            out_specs=_per_t((1, N, 128)),
            out_shape=jax.ShapeDtypeStruct((T, N, 128), f32),
        )(sp, h, dp0, dp1,
          r2(p['gcn_b'][i]), r2(p['gbn_g'][i]), r2(p['gbn_b'][i]))
        if i < 2:
            a = pl.pallas_call(
                _table_body,
                grid=(T,),
                in_specs=[_per_t((1, N, 128)), _full((N, 1)), _full((N, 1)),
                          _full((128, 128))],
                out_specs=_per_t((1, N, 128)),
                out_shape=jax.ShapeDtypeStruct((T, N, 128), f32),
            )(h, dp0, dp1, p['gcn_w'][i + 1])

    H = pl.pallas_call(
        _postmlp_body,
        grid=(T,),
        in_specs=[_per_t((1, N, 128)), _full((128, 128)),
                  _full((1, 128)), _full((1, 128)), _full((1, 128))],
        out_specs=_per_t((1, N, 128)),
        out_shape=jax.ShapeDtypeStruct((T, N, 128), f32),
    )(h, p['post_w'], r2(p['post_b']), r2(p['post_g']), r2(p['post_be']))

    out = pl.pallas_call(
        _gru_body,
        grid=(T,),
        in_specs=[
            _per_t((1, N, 128)),
            _full((384, 128)), _full((384, 128)), _full((1, 384)), _full((1, 384)),
            _full((256, 128)), _full((1, 256)), _full((1, 256)), _full((1, 256)),
            _full((128, 256)), _full((1, 128)),
        ],
        out_specs=_full((N, 128)),
        out_shape=jax.ShapeDtypeStruct((N, 128), f32),
        scratch_shapes=[pltpu.VMEM((N, 128), f32)],
    )(H, p['gru_wih'], p['gru_whh'], r2(p['gru_bih']), r2(p['gru_bhh']),
      p['cls_w1'], r2(p['cls_b1']), r2(p['cls_g']), r2(p['cls_be']),
      jnp.pad(p['cls_w2'], ((0, 128 - p['cls_w2'].shape[0]), (0, 0))),
      r2(jnp.pad(p['cls_b2'], (0, 128 - p['cls_b2'].shape[0]))))

    return out[:, :p['cls_b2'].shape[0]]


# speedup vs baseline: 10.1868x; 1.0004x over previous
import functools

import jax
import jax.numpy as jnp
from jax import lax
from jax.experimental import pallas as pl
from jax.experimental.pallas import tpu as pltpu
from jax.experimental.pallas import tpu_sc as plsc

N = 10000
T = 8
E = 320000
E_SL = E + N
NTILES = 32
SUB = 16
CHUNK = 128
NCH = 82
EPT = NCH * CHUNK
E_PAD = EPT * NTILES
E_ALL = E_PAD + CHUNK
NP = 10240
ROWS_PER_SUB = NP // SUB


EW_OFF = 2 * CHUNK
EROW = EW_OFF + 16 * CHUNK


def _make_spmm(nt):
    mesh = plsc.VectorSubcoreMesh(core_axis_name="c", subcore_axis_name="s")

    @functools.partial(
        pl.kernel,
        mesh=mesh,
        out_type=jax.ShapeDtypeStruct((2, nt, NP, 128), jnp.float32),
        scratch_types=[
            pltpu.VMEM((EROW,), jnp.int32),
            pltpu.VMEM((EROW,), jnp.int32),
            pltpu.VMEM((CHUNK,), jnp.int32),
            pltpu.VMEM((CHUNK,), jnp.int32),
            pltpu.VMEM((CHUNK,), jnp.int32),
            pltpu.VMEM((CHUNK,), jnp.int32),
            pltpu.VMEM((CHUNK, 128), jnp.float32),
            pltpu.VMEM((CHUNK, 128), jnp.float32),
            pltpu.VMEM_SHARED((NP, 128), jnp.float32),
            pltpu.SemaphoreType.DMA,
            pltpu.SemaphoreType.DMA,
            pltpu.SemaphoreType.DMA,
            pltpu.SemaphoreType.DMA,
            pltpu.SemaphoreType.DMA,
            pltpu.SemaphoreType.DMA,
        ],
    )
    def spmm(table_hbm, edata_hbm, zeros_hbm, out_hbm,
             ebA, ebB, riA, riB, sciA, sciB, rowsA, rowsB, acc,
             sem_gA, sem_gB, sem_iA, sem_iB, sem_sA, sem_sB):
        c = lax.axis_index("c")
        s = lax.axis_index("s")
        wid = s * 2 + c
        cbase = wid * NCH

        def mul(rows_v, eb):
            def edge_body(e, carry2):
                bc = lax.bitcast_convert_type(
                    eb[pl.ds(EW_OFF + e * 16, 16)], jnp.float32)
                for g in range(8):
                    rows_v[e, pl.ds(g * 16, 16)] = (
                        rows_v[e, pl.ds(g * 16, 16)] * bc)
                return carry2
            lax.fori_loop(0, CHUNK, edge_body, 0, unroll=4)

        def adjust(t, eb, ri):
            off = jnp.full((16,), t * N, jnp.int32)
            for g in range(CHUNK // 16):
                ri[pl.ds(g * 16, 16)] = eb[pl.ds(g * 16, 16)] + off

        def step(t, j, cur, nxt, first, last):
            (eb0, ri0, sci0, rows0, sem_g0, sem_i0, sem_s0) = cur
            (eb1, ri1, sci1, rows1, sem_g1, sem_i1, sem_s1) = nxt
            if not first:
                pltpu.make_async_copy(rows1, acc.at[sci1], sem_s1).wait()
                pltpu.make_async_copy(edata_hbm.at[0], eb1, sem_i1).wait()
            if not last:
                adjust(t, eb1, ri1)
                pltpu.async_copy(table_hbm.at[ri1], rows1, sem_g1)
            pltpu.make_async_copy(table_hbm.at[ri0], rows0, sem_g0).wait()
            for g in range(CHUNK // 16):
                sci0[pl.ds(g * 16, 16)] = eb0[pl.ds(CHUNK + g * 16, 16)]
            mul(rows0, eb0)
            pltpu.async_copy(rows0, acc.at[sci0], sem_s0, add=True)
            if not last:
                pltpu.async_copy(edata_hbm.at[cbase + j + 2], eb0, sem_i0)

        bufA = (ebA, riA, sciA, rowsA, sem_gA, sem_iA, sem_sA)
        bufB = (ebB, riB, sciB, rowsB, sem_gB, sem_iB, sem_sB)

        def t_body(t, carry):
            pltpu.sync_copy(zeros_hbm, acc.at[pl.ds(s * ROWS_PER_SUB, ROWS_PER_SUB)])
            plsc.subcore_barrier()

            pltpu.sync_copy(edata_hbm.at[cbase], ebA)
            adjust(t, ebA, riA)
            pltpu.async_copy(table_hbm.at[riA], rowsA, sem_gA)
            pltpu.sync_copy(edata_hbm.at[cbase + 1], ebB)

            step(t, 0, bufA, bufB, True, False)
            step(t, 1, bufB, bufA, False, False)

            def pair_body(k, carry2):
                step(t, 2 * k, bufA, bufB, False, False)
                step(t, 2 * k + 1, bufB, bufA, False, False)
                return carry2
            lax.fori_loop(1, NCH // 2 - 1, pair_body, 0)
            step(t, NCH - 2, bufA, bufB, False, False)
            step(t, NCH - 1, bufB, bufA, False, True)
            pltpu.make_async_copy(rowsB, acc.at[sciB], sem_sB).wait()

            plsc.subcore_barrier()
            pltpu.sync_copy(
                acc.at[pl.ds(s * ROWS_PER_SUB, ROWS_PER_SUB)],
                out_hbm.at[c, t, pl.ds(s * ROWS_PER_SUB, ROWS_PER_SUB)])
            plsc.subcore_barrier()
            return carry

        lax.fori_loop(0, nt, t_body, 0)

    return spmm


_spmm_T = _make_spmm(T)
_spmm_1 = _make_spmm(1)


def _bn_relu(y, g, b, n_rows):
    mu = jnp.sum(y, axis=0, keepdims=True) / n_rows
    var = jnp.sum((y - mu) ** 2, axis=0, keepdims=True) / n_rows
    return jnp.maximum(g * (y - mu) / jnp.sqrt(var + 1e-5) + b, 0.0)


def _dis_from_parts(p0, p1):
    return lax.rsqrt(p0 + p1)


def _pre_body(x_ref, p0_ref, p1_ref, w1_ref, b1_ref, g1_ref, be1_ref,
              w2_ref, b2_ref, g2_ref, be2_ref, gw_ref,
              h_ref, a_ref):
    x = x_ref[0]
    h = jnp.dot(x, w1_ref[...].T, preferred_element_type=jnp.float32) + b1_ref[...]
    h = _bn_relu(h, g1_ref[...], be1_ref[...], N)
    h = jnp.dot(h, w2_ref[...].T, preferred_element_type=jnp.float32) + b2_ref[...]
    h = _bn_relu(h, g2_ref[...], be2_ref[...], N)
    h_ref[0] = h
    dis = _dis_from_parts(p0_ref[...], p1_ref[...])
    a_ref[0] = dis * jnp.dot(h, gw_ref[...].T, preferred_element_type=jnp.float32)


def _resid_body(sp_ref, h_ref, p0_ref, p1_ref, gb_ref, g_ref, be_ref,
                h_out_ref):
    dis = _dis_from_parts(p0_ref[...], p1_ref[...])
    srow = sp_ref[0, 0, :N] + sp_ref[1, 0, :N]
    hn = dis * srow + gb_ref[...]
    hn = _bn_relu(hn, g_ref[...], be_ref[...], N)
    h_out_ref[0] = h_ref[0] + hn


def _table_body(h_ref, p0_ref, p1_ref, w_ref, a_ref):
    dis = _dis_from_parts(p0_ref[...], p1_ref[...])
    a_ref[0] = dis * jnp.dot(h_ref[0], w_ref[...].T,
                             preferred_element_type=jnp.float32)


def _postmlp_body(h_ref, w_ref, b_ref, g_ref, be_ref, out_ref):
    y = jnp.dot(h_ref[0], w_ref[...].T, preferred_element_type=jnp.float32)
    out_ref[0] = _bn_relu(y + b_ref[...], g_ref[...], be_ref[...], N)


def _gru_body(H_ref, wih_ref, whh_ref, bih_ref, bhh_ref,
              w1_ref, b1_ref, g_ref, be_ref, w2_ref, b2_ref,
              out_ref, hstate):
    t = pl.program_id(0)

    @pl.when(t == 0)
    def _():
        hstate[...] = jnp.zeros((N, 128), jnp.float32)

    h = H_ref[0]
    gi = jnp.dot(h, wih_ref[...].T, preferred_element_type=jnp.float32) + bih_ref[...]
    gh = jnp.dot(hstate[...], whh_ref[...].T, preferred_element_type=jnp.float32) + bhh_ref[...]
    r = jax.nn.sigmoid(gi[:, :128] + gh[:, :128])
    z = jax.nn.sigmoid(gi[:, 128:256] + gh[:, 128:256])
    n_ = jnp.tanh(gi[:, 256:] + r * gh[:, 256:])
    hs = (1.0 - z) * n_ + z * hstate[...]
    hstate[...] = hs

    @pl.when(t == T - 1)
    def _():
        y = jnp.dot(hs, w1_ref[...].T, preferred_element_type=jnp.float32) + b1_ref[...]
        y = _bn_relu(y, g_ref[...], be_ref[...], N)
        out_ref[...] = jnp.dot(y, w2_ref[...].T, preferred_element_type=jnp.float32) + b2_ref[...]


def _full(shape):
    return pl.BlockSpec(shape, lambda t: tuple(0 for _ in shape))


def _per_t(shape):
    return pl.BlockSpec(shape, lambda t: (t,) + tuple(0 for _ in shape[1:]))


def kernel(x, edge_weight, params, edge_index):
    p = params
    f32 = jnp.float32

    npad = E_ALL - E_SL
    sl = jnp.arange(N, dtype=jnp.int32)
    row = jnp.concatenate(
        [edge_index[0], sl, (jnp.arange(npad, dtype=jnp.int32) * 37) % N])
    col = jnp.concatenate([edge_index[1], sl, jnp.zeros((npad,), jnp.int32)])
    ew = jnp.concatenate(
        [edge_weight.astype(f32), jnp.ones((N,), f32), jnp.zeros((npad,), f32)])
    ew_wide = jnp.broadcast_to(ew[:, None], (E_ALL, 16))
    zeros_sub = jnp.zeros((ROWS_PER_SUB, 128), f32)
    ncht = E_ALL // CHUNK
    edata = jnp.concatenate(
        [row.reshape(ncht, CHUNK), col.reshape(ncht, CHUNK),
         jax.lax.bitcast_convert_type(ew_wide, jnp.int32).reshape(ncht, 16 * CHUNK)],
        axis=1)

    degp = _spmm_1(jnp.ones((N, 128), f32), edata, zeros_sub)
    dp0 = degp[0, 0, :N, 0:1]
    dp1 = degp[1, 0, :N, 0:1]

    r2 = lambda v: v.reshape(1, -1)

    h0, a0 = pl.pallas_call(
        _pre_body,
        grid=(T,),
        in_specs=[
            _per_t((1, N, 128)),
            _full((N, 1)), _full((N, 1)),
            _full((256, 128)), _full((1, 256)), _full((1, 256)), _full((1, 256)),
            _full((128, 256)), _full((1, 128)), _full((1, 128)), _full((1, 128)),
            _full((128, 128)),
        ],
        out_specs=[_per_t((1, N, 128)), _per_t((1, N, 128))],
        out_shape=[jax.ShapeDtypeStruct((T, N, 128), f32),
                   jax.ShapeDtypeStruct((T, N, 128), f32)],
    )(x, dp0, dp1,
      p['pre_w1'], r2(p['pre_b1']), r2(p['pre_g1']), r2(p['pre_be1']),
      p['pre_w2'], r2(p['pre_b2']), r2(p['pre_g2']), r2(p['pre_be2']),
      p['gcn_w'][0])

    h, a = h0, a0
    for i in range(3):
        sp = _spmm_T(a.reshape(T * N, 128), edata, zeros_sub)
        h = pl.pallas_call(
            _resid_body,
            grid=(T,),
            in_specs=[
                pl.BlockSpec((2, 1, NP, 128), lambda t: (0, t, 0, 0)),
                _per_t((1, N, 128)),
                _full((N, 1)), _full((N, 1)),
                _full((1, 128)), _full((1, 128)), _full((1, 128)),
            ],
            out_specs=_per_t((1, N, 128)),
            out_shape=jax.ShapeDtypeStruct((T, N, 128), f32),
        )(sp, h, dp0, dp1,
          r2(p['gcn_b'][i]), r2(p['gbn_g'][i]), r2(p['gbn_b'][i]))
        if i < 2:
            a = pl.pallas_call(
                _table_body,
                grid=(T,),
                in_specs=[_per_t((1, N, 128)), _full((N, 1)), _full((N, 1)),
                          _full((128, 128))],
                out_specs=_per_t((1, N, 128)),
                out_shape=jax.ShapeDtypeStruct((T, N, 128), f32),
            )(h, dp0, dp1, p['gcn_w'][i + 1])

    H = pl.pallas_call(
        _postmlp_body,
        grid=(T,),
        in_specs=[_per_t((1, N, 128)), _full((128, 128)),
                  _full((1, 128)), _full((1, 128)), _full((1, 128))],
        out_specs=_per_t((1, N, 128)),
        out_shape=jax.ShapeDtypeStruct((T, N, 128), f32),
    )(h, p['post_w'], r2(p['post_b']), r2(p['post_g']), r2(p['post_be']))

    out = pl.pallas_call(
        _gru_body,
        grid=(T,),
        in_specs=[
            _per_t((1, N, 128)),
            _full((384, 128)), _full((384, 128)), _full((1, 384)), _full((1, 384)),
            _full((256, 128)), _full((1, 256)), _full((1, 256)), _full((1, 256)),
            _full((128, 256)), _full((1, 128)),
        ],
        out_specs=_full((N, 128)),
        out_shape=jax.ShapeDtypeStruct((N, 128), f32),
        scratch_shapes=[pltpu.VMEM((N, 128), f32)],
    )(H, p['gru_wih'], p['gru_whh'], r2(p['gru_bih']), r2(p['gru_bhh']),
      p['cls_w1'], r2(p['cls_b1']), r2(p['cls_g']), r2(p['cls_be']),
      jnp.pad(p['cls_w2'], ((0, 128 - p['cls_w2'].shape[0]), (0, 0))),
      r2(jnp.pad(p['cls_b2'], (0, 128 - p['cls_b2'].shape[0]))))

    return out[:, :p['cls_b2'].shape[0]]
